# baseline (speedup 1.0000x reference)
import functools

import jax
import jax.numpy as jnp
from jax import lax
from jax.experimental import pallas as pl
from jax.experimental.pallas import tpu as pltpu
from jax.experimental.pallas import tpu_sc as plsc

_N = 10000
_E = 160000
_D = 128
_H = 64
_C = 16
_T = 1000
_WD = 5e-4

_TBR = 512
_TBC = 1024
_NP = 10240
_GI = _NP // _TBR
_GJ = _NP // _TBC

_TP = 1024
_RT = 128


def _l2n(x):
    n = jnp.sqrt(jnp.sum(x * x, axis=1, keepdims=True))
    return x / jnp.maximum(n, 1e-12)


def _nxn_body(g_ref, ht_ref, rowp_ref, colp_ref):
    s = jax.lax.dot_general(g_ref[...], ht_ref[...], (((1,), (0,)), ((), ())),
                            preferred_element_type=jnp.float32)
    p = jnp.exp(s)
    ones_c = jnp.ones((_TBC, 1), jnp.float32)
    ones_r = jnp.ones((1, _TBR), jnp.float32)
    rowp_ref[...] = jax.lax.dot_general(
        p, ones_c, (((1,), (0,)), ((), ())),
        preferred_element_type=jnp.float32).reshape(1, _TBR, 1)
    colp_ref[...] = jax.lax.dot_general(
        ones_r, p, (((1,), (0,)), ((), ())),
        preferred_element_type=jnp.float32).reshape(1, 1, _TBC)


def _nxn_reductions(g2p, htp):
    rowp, colp = pl.pallas_call(
        _nxn_body,
        grid=(_GI, _GJ),
        in_specs=[
            pl.BlockSpec((_TBR, _C), lambda i, j: (i, 0)),
            pl.BlockSpec((_C, _TBC), lambda i, j: (0, j)),
        ],
        out_specs=[
            pl.BlockSpec((1, _TBR, 1), lambda i, j: (j, i, 0)),
            pl.BlockSpec((1, 1, _TBC), lambda i, j: (i, 0, j)),
        ],
        out_shape=[
            jax.ShapeDtypeStruct((_GJ, _NP, 1), jnp.float32),
            jax.ShapeDtypeStruct((_GI, 1, _NP), jnp.float32),
        ],
    )(g2p, htp)
    pad = _NP - _N
    rowsum = jnp.sum(rowp, axis=0)[:_N, 0] - pad
    colsum = jnp.sum(colp, axis=(0, 1))[:_N] - pad
    return rowsum, colsum


def _sup_body(h1_ref, h2t_ref, intra_ref, intrat_ref,
              sprow_ref, rowtot_ref, spcol_ref, coltot_ref):
    i = pl.program_id(0)
    s = jax.lax.dot_general(h1_ref[...], h2t_ref[...], (((1,), (0,)), ((), ())),
                            preferred_element_type=jnp.float32)
    p = jnp.exp(s)
    sprow_ref[...] = jnp.sum(p * intra_ref[...], axis=1, keepdims=True)
    rowtot_ref[...] = jnp.sum(p, axis=1, keepdims=True)

    @pl.when(i == 0)
    def _init():
        spcol_ref[...] = jnp.zeros_like(spcol_ref)
        coltot_ref[...] = jnp.zeros_like(coltot_ref)

    spcol_ref[...] += jnp.sum(p * intrat_ref[...], axis=0, keepdims=True)
    coltot_ref[...] += jnp.sum(p, axis=0, keepdims=True)


def _sup_reductions(h1p2, h2tp, intrap, intratp):
    grid = (_TP // _RT,)
    sprow, rowtot, spcol, coltot = pl.pallas_call(
        _sup_body,
        grid=grid,
        in_specs=[
            pl.BlockSpec((_RT, _C), lambda i: (i, 0)),
            pl.BlockSpec((_C, _TP), lambda i: (0, 0)),
            pl.BlockSpec((_RT, _TP), lambda i: (i, 0)),
            pl.BlockSpec((_RT, _TP), lambda i: (i, 0)),
        ],
        out_specs=[
            pl.BlockSpec((_RT, 1), lambda i: (i, 0)),
            pl.BlockSpec((_RT, 1), lambda i: (i, 0)),
            pl.BlockSpec((1, _TP), lambda i: (0, 0)),
            pl.BlockSpec((1, _TP), lambda i: (0, 0)),
        ],
        out_shape=[
            jax.ShapeDtypeStruct((_TP, 1), jnp.float32),
            jax.ShapeDtypeStruct((_TP, 1), jnp.float32),
            jax.ShapeDtypeStruct((1, _TP), jnp.float32),
            jax.ShapeDtypeStruct((1, _TP), jnp.float32),
        ],
    )(h1p2, h2tp, intrap, intratp)
    padt = _TP - _T
    return (sprow[:_T, 0], rowtot[:_T, 0] - padt,
            spcol[0, :_T], coltot[0, :_T] - padt)


_NW = 32
_EP = 160256
_EB = _EP // _NW


def _edge_logits_sc(u1, v1, u2, v2, epi, epj):
    mesh = plsc.VectorSubcoreMesh(core_axis_name="c", subcore_axis_name="s")

    @functools.partial(
        pl.kernel, mesh=mesh,
        out_type=[jax.ShapeDtypeStruct((_EP,), jnp.float32),
                  jax.ShapeDtypeStruct((_EP,), jnp.float32)],
        scratch_types=[pltpu.VMEM((_EB,), jnp.int32)] * 2
        + [pltpu.VMEM((_EB,), jnp.float32)] * 4
        + [pltpu.SemaphoreType.DMA],
    )
    def ek(u1_h, v1_h, u2_h, v2_h, epi_h, epj_h, s1_h, s2_h,
           ei_v, ej_v, a1_v, b1_v, a2_v, b2_v, sem):
        wid = lax.axis_index("s") * 2 + lax.axis_index("c")
        base = wid * _EB
        pltpu.sync_copy(epi_h.at[pl.ds(base, _EB)], ei_v)
        pltpu.sync_copy(epj_h.at[pl.ds(base, _EB)], ej_v)
        pltpu.async_copy(u1_h.at[ei_v], a1_v, sem)
        pltpu.async_copy(v1_h.at[ej_v], b1_v, sem)
        pltpu.async_copy(u2_h.at[ei_v], a2_v, sem)
        last = pltpu.async_copy(v2_h.at[ej_v], b2_v, sem)
        last.wait()
        last.wait()
        last.wait()
        last.wait()

        def body(k, carry):
            sl = pl.ds(k * 16, 16)
            a1_v[sl] = a1_v[sl] + b1_v[sl]
            a2_v[sl] = a2_v[sl] + b2_v[sl]
            return carry

        lax.fori_loop(0, _EB // 16, body, 0)
        pltpu.sync_copy(a1_v, s1_h.at[pl.ds(base, _EB)])
        pltpu.sync_copy(a2_v, s2_h.at[pl.ds(base, _EB)])

    return ek(u1, v1, u2, v2, epi, epj)


_NCH = 40
_CB = 128
_EP2 = 32 * _NCH * _CB
_RPT = 624


def _spmm_sc(x, src3, dst3, w3, k):
    mesh = plsc.VectorSubcoreMesh(core_axis_name="c", subcore_axis_name="s")

    @functools.partial(
        pl.kernel, mesh=mesh,
        out_type=jax.ShapeDtypeStruct((2, _N, k), jnp.float32),
        scratch_types=[
            pltpu.VMEM((_NCH, _CB), jnp.int32),
            pltpu.VMEM((_NCH, _CB), jnp.int32),
            pltpu.VMEM((_NCH, _CB), jnp.float32),
            pltpu.VMEM((2, _CB, k), jnp.float32),
            pltpu.VMEM_SHARED((_N, k), jnp.float32),
            pltpu.SemaphoreType.DMA,
            pltpu.SemaphoreType.DMA,
            pltpu.SemaphoreType.DMA,
            pltpu.SemaphoreType.DMA,
        ],
    )
    def sk(x_h, src_h, dst_h, w_h, z_h, out_h,
           src_v, dst_v, w_v, rows2_v, acc_sh,
           g0, g1, s0, s1):
        cid = lax.axis_index("c")
        sid = lax.axis_index("s")
        wid = cid * 16 + sid
        rbase = sid * _RPT
        pltpu.sync_copy(z_h.at[pl.ds(rbase, _RPT)],
                        acc_sh.at[pl.ds(rbase, _RPT)])

        @pl.when(sid == 15)
        def _zero_tail():
            pltpu.sync_copy(z_h.at[pl.ds(16 * _RPT, _N - 16 * _RPT)],
                            acc_sh.at[pl.ds(16 * _RPT, _N - 16 * _RPT)])
        pltpu.sync_copy(src_h.at[wid], src_v)
        pltpu.sync_copy(dst_h.at[wid], dst_v)
        pltpu.sync_copy(w_h.at[wid], w_v)
        plsc.subcore_barrier()

        gsems = (g0, g1)
        ssems = (s0, s1)

        def scale(rv, ci):
            def grp(g, c2):
                wv = w_v[ci, pl.ds(g * 16, 16)]
                for l in range(16):
                    wb = jnp.take(wv, jnp.full((16,), l, jnp.int32))
                    e = g * 16 + l
                    for kk in range(k // 16):
                        sl = pl.ds(kk * 16, 16)
                        rv[e, sl] = rv[e, sl] * wb
                return c2

            lax.fori_loop(0, _CB // 16, grp, 0)

        pltpu.async_copy(x_h.at[src_v.at[0]], rows2_v.at[0], gsems[0])

        def pipe(g, carry):
            for b in range(2):
                ci = 2 * g + b
                bn = 1 - b
                if b == 0:
                    @pl.when(g > 0)
                    def _dr0():
                        pltpu.make_async_copy(z_h.at[pl.ds(0, _CB)],
                                              rows2_v.at[bn], ssems[bn]).wait()

                    pltpu.async_copy(x_h.at[src_v.at[ci + 1]],
                                     rows2_v.at[bn], gsems[bn])
                else:
                    pltpu.make_async_copy(z_h.at[pl.ds(0, _CB)],
                                          rows2_v.at[bn], ssems[bn]).wait()

                    @pl.when(g < (_NCH // 2) - 1)
                    def _ng():
                        pltpu.async_copy(x_h.at[src_v.at[ci + 1]],
                                         rows2_v.at[bn], gsems[bn])

                pltpu.make_async_copy(x_h.at[src_v.at[ci]],
                                      rows2_v.at[b], gsems[b]).wait()
                scale(rows2_v.at[b], ci)
                pltpu.async_copy(rows2_v.at[b], acc_sh.at[dst_v.at[ci]],
                                 ssems[b], add=True)
            return carry

        lax.fori_loop(0, _NCH // 2, pipe, 0)
        pltpu.make_async_copy(z_h.at[pl.ds(0, _CB)],
                              rows2_v.at[(_NCH - 1) % 2],
                              ssems[(_NCH - 1) % 2]).wait()
        plsc.subcore_barrier()
        pltpu.sync_copy(acc_sh.at[pl.ds(rbase, _RPT)],
                        out_h.at[cid, pl.ds(rbase, _RPT)])

        @pl.when(sid == 15)
        def _out_tail():
            pltpu.sync_copy(acc_sh.at[pl.ds(16 * _RPT, _N - 16 * _RPT)],
                            out_h.at[cid, pl.ds(16 * _RPT, _N - 16 * _RPT)])

    parts = sk(x, src3, dst3, w3, jnp.zeros((_N, k), jnp.float32))
    return parts[0] + parts[1]


def kernel(features, edge_src, edge_dst, edge_w, labels, mask,
           edge_pos_i, edge_pos_j, train_idx, mat01_intra, mat01_inter,
           W0, b0, W1, b1, Wh0, bh0, Wh1, bh1, Wc, bc):
    epad = _EP2 - _E
    src3 = jnp.pad(edge_src.astype(jnp.int32), (0, epad)).reshape(32, _NCH, _CB)
    dst3 = jnp.pad(edge_dst.astype(jnp.int32), (0, epad)).reshape(32, _NCH, _CB)
    w3 = jnp.pad(edge_w, (0, epad)).reshape(32, _NCH, _CB)
    sfeat = _spmm_sc(features, src3, dst3, w3, _D)
    h0 = jax.nn.relu(sfeat @ W0 + b0)
    hg0 = jax.nn.relu(sfeat @ Wh0 + bh0)
    s1cat = _spmm_sc(jnp.concatenate([h0, hg0], axis=1), src3, dst3, w3, _D)
    cv_gcn = _l2n(s1cat[:, :_H] @ W1 + b1)
    cv_hg = _l2n(s1cat[:, _H:] @ Wh1 + bh1)
    outputs = _l2n(0.6 * cv_gcn + 0.4 * cv_hg)

    m = mask / jnp.mean(mask)
    logp = jax.nn.log_softmax(outputs, axis=1)
    loss_q = jnp.mean(-(labels * logp).sum(axis=1) * m)

    a = Wc[:_C, 0]
    b = Wc[_C:, 0]
    u1 = jnp.pad(cv_gcn @ a + bc[0], (0, _NP - _N))
    v1 = jnp.pad(cv_hg @ b, (0, _NP - _N))
    u2 = jnp.pad(cv_hg @ a + bc[0], (0, _NP - _N))
    v2 = jnp.pad(cv_gcn @ b, (0, _NP - _N))
    epi = jnp.pad(edge_pos_i.astype(jnp.int32), (0, _EP - _E))
    epj = jnp.pad(edge_pos_j.astype(jnp.int32), (0, _EP - _E))
    s1e, s2e = _edge_logits_sc(u1, v1, u2, v2, epi, epj)
    p1 = -jnp.mean(jnp.log(jax.nn.sigmoid(s1e[:_E])))
    p2 = -jnp.mean(jnp.log(jax.nn.sigmoid(s2e[:_E])))
    p_e_xy = p1 + p2

    pad = _NP - _N
    g2p = jnp.pad(2.0 * cv_gcn, ((0, pad), (0, 0)))
    htp = jnp.pad(cv_hg.T, ((0, 0), (0, pad)))
    rowsum, colsum = _nxn_reductions(g2p, htp)
    d = jnp.exp(2.0 * jnp.sum(cv_gcn * cv_hg, axis=1))
    pn1 = d / (rowsum / _N)
    pn2 = d / (colsum / _N)
    closs = -0.9 * jnp.mean(jnp.log(jnp.concatenate([pn1, pn2], axis=0)))

    h1s = cv_gcn[train_idx]
    h2s = cv_hg[train_idx]
    padt = _TP - _T
    h1p2 = jnp.pad(2.0 * h1s, ((0, padt), (0, 0)))
    h2tp = jnp.pad(h2s.T, ((0, 0), (0, padt)))
    intrap = jnp.pad(mat01_intra, ((0, padt), (0, padt)))
    intratp = jnp.pad(mat01_intra.T, ((0, padt), (0, padt)))
    sprow, rowtot, spcol, coltot = _sup_reductions(h1p2, h2tp, intrap, intratp)
    rowsum01 = jnp.sum(mat01_intra, axis=1)
    sup1 = (sprow / rowsum01) / (rowtot / (_T - 1))
    sup2 = (spcol / rowsum01) / (coltot / (_T - 1))
    closs = closs - 0.9 * jnp.mean(jnp.log(jnp.concatenate([sup1, sup2], axis=0)))

    total = loss_q + 0.4 * p_e_xy + closs
    for v in (W0, b0, W1, b1, Wc, bc):
        total = total + _WD * 0.5 * jnp.sum(v ** 2)

    acc = jnp.mean((jnp.argmax(outputs, axis=1) ==
                    jnp.argmax(labels, axis=1)).astype(jnp.float32) * m)
    return (outputs, total, acc)


# speedup vs baseline: 1.0379x; 1.0379x over previous
import functools

import jax
import jax.numpy as jnp
from jax import lax
from jax.experimental import pallas as pl
from jax.experimental.pallas import tpu as pltpu
from jax.experimental.pallas import tpu_sc as plsc

_N = 10000
_E = 160000
_D = 128
_H = 64
_C = 16
_T = 1000
_WD = 5e-4

_TBR = 512
_TBC = 1024
_NP = 10240
_GI = _NP // _TBR
_GJ = _NP // _TBC

_TP = 1024
_RT = 128


def _l2n(x):
    n = jnp.sqrt(jnp.sum(x * x, axis=1, keepdims=True))
    return x / jnp.maximum(n, 1e-12)


def _nxn_body(g_ref, ht_ref, rowp_ref, colp_ref):
    s = jax.lax.dot_general(g_ref[...], ht_ref[...], (((1,), (0,)), ((), ())),
                            preferred_element_type=jnp.float32)
    p = jnp.exp(s)
    rowp_ref[...] = jnp.sum(p, axis=1, keepdims=True).reshape(1, _TBR, 1)
    colp_ref[...] = jnp.sum(p, axis=0, keepdims=True).reshape(1, 1, _TBC)


def _nxn_reductions(g2p, htp):
    rowp, colp = pl.pallas_call(
        _nxn_body,
        grid=(_GI, _GJ),
        in_specs=[
            pl.BlockSpec((_TBR, _C), lambda i, j: (i, 0)),
            pl.BlockSpec((_C, _TBC), lambda i, j: (0, j)),
        ],
        out_specs=[
            pl.BlockSpec((1, _TBR, 1), lambda i, j: (j, i, 0)),
            pl.BlockSpec((1, 1, _TBC), lambda i, j: (i, 0, j)),
        ],
        out_shape=[
            jax.ShapeDtypeStruct((_GJ, _NP, 1), jnp.float32),
            jax.ShapeDtypeStruct((_GI, 1, _NP), jnp.float32),
        ],
    )(g2p, htp)
    pad = _NP - _N
    rowsum = jnp.sum(rowp, axis=0)[:_N, 0] - pad
    colsum = jnp.sum(colp, axis=(0, 1))[:_N] - pad
    return rowsum, colsum


def _sup_body(h1_ref, h2t_ref, intra_ref, intrat_ref,
              sprow_ref, rowtot_ref, spcol_ref, coltot_ref):
    i = pl.program_id(0)
    s = jax.lax.dot_general(h1_ref[...], h2t_ref[...], (((1,), (0,)), ((), ())),
                            preferred_element_type=jnp.float32)
    p = jnp.exp(s)
    sprow_ref[...] = jnp.sum(p * intra_ref[...], axis=1, keepdims=True)
    rowtot_ref[...] = jnp.sum(p, axis=1, keepdims=True)

    @pl.when(i == 0)
    def _init():
        spcol_ref[...] = jnp.zeros_like(spcol_ref)
        coltot_ref[...] = jnp.zeros_like(coltot_ref)

    spcol_ref[...] += jnp.sum(p * intrat_ref[...], axis=0, keepdims=True)
    coltot_ref[...] += jnp.sum(p, axis=0, keepdims=True)


def _sup_reductions(h1p2, h2tp, intrap, intratp):
    grid = (_TP // _RT,)
    sprow, rowtot, spcol, coltot = pl.pallas_call(
        _sup_body,
        grid=grid,
        in_specs=[
            pl.BlockSpec((_RT, _C), lambda i: (i, 0)),
            pl.BlockSpec((_C, _TP), lambda i: (0, 0)),
            pl.BlockSpec((_RT, _TP), lambda i: (i, 0)),
            pl.BlockSpec((_RT, _TP), lambda i: (i, 0)),
        ],
        out_specs=[
            pl.BlockSpec((_RT, 1), lambda i: (i, 0)),
            pl.BlockSpec((_RT, 1), lambda i: (i, 0)),
            pl.BlockSpec((1, _TP), lambda i: (0, 0)),
            pl.BlockSpec((1, _TP), lambda i: (0, 0)),
        ],
        out_shape=[
            jax.ShapeDtypeStruct((_TP, 1), jnp.float32),
            jax.ShapeDtypeStruct((_TP, 1), jnp.float32),
            jax.ShapeDtypeStruct((1, _TP), jnp.float32),
            jax.ShapeDtypeStruct((1, _TP), jnp.float32),
        ],
    )(h1p2, h2tp, intrap, intratp)
    padt = _TP - _T
    return (sprow[:_T, 0], rowtot[:_T, 0] - padt,
            spcol[0, :_T], coltot[0, :_T] - padt)


_NW = 32
_EP = 160256
_EB = _EP // _NW


def _edge_logits_sc(u1, v1, u2, v2, epi, epj):
    mesh = plsc.VectorSubcoreMesh(core_axis_name="c", subcore_axis_name="s")

    @functools.partial(
        pl.kernel, mesh=mesh,
        out_type=[jax.ShapeDtypeStruct((_EP,), jnp.float32),
                  jax.ShapeDtypeStruct((_EP,), jnp.float32)],
        scratch_types=[pltpu.VMEM((_EB,), jnp.int32)] * 2
        + [pltpu.VMEM((_EB,), jnp.float32)] * 4
        + [pltpu.SemaphoreType.DMA],
    )
    def ek(u1_h, v1_h, u2_h, v2_h, epi_h, epj_h, s1_h, s2_h,
           ei_v, ej_v, a1_v, b1_v, a2_v, b2_v, sem):
        wid = lax.axis_index("s") * 2 + lax.axis_index("c")
        base = wid * _EB
        pltpu.sync_copy(epi_h.at[pl.ds(base, _EB)], ei_v)
        pltpu.sync_copy(epj_h.at[pl.ds(base, _EB)], ej_v)
        pltpu.async_copy(u1_h.at[ei_v], a1_v, sem)
        pltpu.async_copy(v1_h.at[ej_v], b1_v, sem)
        pltpu.async_copy(u2_h.at[ei_v], a2_v, sem)
        last = pltpu.async_copy(v2_h.at[ej_v], b2_v, sem)
        last.wait()
        last.wait()
        last.wait()
        last.wait()

        def body(k, carry):
            sl = pl.ds(k * 16, 16)
            a1_v[sl] = a1_v[sl] + b1_v[sl]
            a2_v[sl] = a2_v[sl] + b2_v[sl]
            return carry

        lax.fori_loop(0, _EB // 16, body, 0)
        pltpu.sync_copy(a1_v, s1_h.at[pl.ds(base, _EB)])
        pltpu.sync_copy(a2_v, s2_h.at[pl.ds(base, _EB)])

    return ek(u1, v1, u2, v2, epi, epj)


_NCH = 40
_CB = 128
_EP2 = 32 * _NCH * _CB
_RPT = 624


def _spmm_sc(x, src3, dst3, w3, k):
    mesh = plsc.VectorSubcoreMesh(core_axis_name="c", subcore_axis_name="s")

    @functools.partial(
        pl.kernel, mesh=mesh,
        out_type=jax.ShapeDtypeStruct((2, _N, k), jnp.float32),
        scratch_types=[
            pltpu.VMEM((_NCH, _CB), jnp.int32),
            pltpu.VMEM((_NCH, _CB), jnp.int32),
            pltpu.VMEM((_NCH, _CB), jnp.float32),
            pltpu.VMEM((2, _CB, k), jnp.float32),
            pltpu.VMEM_SHARED((_N, k), jnp.float32),
            pltpu.SemaphoreType.DMA,
            pltpu.SemaphoreType.DMA,
            pltpu.SemaphoreType.DMA,
            pltpu.SemaphoreType.DMA,
        ],
    )
    def sk(x_h, src_h, dst_h, w_h, z_h, out_h,
           src_v, dst_v, w_v, rows2_v, acc_sh,
           g0, g1, s0, s1):
        cid = lax.axis_index("c")
        sid = lax.axis_index("s")
        wid = cid * 16 + sid
        rbase = sid * _RPT
        pltpu.sync_copy(z_h.at[pl.ds(rbase, _RPT)],
                        acc_sh.at[pl.ds(rbase, _RPT)])

        @pl.when(sid == 15)
        def _zero_tail():
            pltpu.sync_copy(z_h.at[pl.ds(16 * _RPT, _N - 16 * _RPT)],
                            acc_sh.at[pl.ds(16 * _RPT, _N - 16 * _RPT)])
        pltpu.sync_copy(src_h.at[wid], src_v)
        pltpu.sync_copy(dst_h.at[wid], dst_v)
        pltpu.sync_copy(w_h.at[wid], w_v)
        plsc.subcore_barrier()

        gsems = (g0, g1)
        ssems = (s0, s1)

        def scale(rv, ci):
            def grp(g, c2):
                wv = w_v[ci, pl.ds(g * 16, 16)]
                for l in range(16):
                    wb = jnp.take(wv, jnp.full((16,), l, jnp.int32))
                    e = g * 16 + l
                    for kk in range(k // 16):
                        sl = pl.ds(kk * 16, 16)
                        rv[e, sl] = rv[e, sl] * wb
                return c2

            lax.fori_loop(0, _CB // 16, grp, 0)

        pltpu.async_copy(x_h.at[src_v.at[0]], rows2_v.at[0], gsems[0])

        def pipe(g, carry):
            for b in range(2):
                ci = 2 * g + b
                bn = 1 - b
                if b == 0:
                    @pl.when(g > 0)
                    def _dr0():
                        pltpu.make_async_copy(z_h.at[pl.ds(0, _CB)],
                                              rows2_v.at[bn], ssems[bn]).wait()

                    pltpu.async_copy(x_h.at[src_v.at[ci + 1]],
                                     rows2_v.at[bn], gsems[bn])
                else:
                    pltpu.make_async_copy(z_h.at[pl.ds(0, _CB)],
                                          rows2_v.at[bn], ssems[bn]).wait()

                    @pl.when(g < (_NCH // 2) - 1)
                    def _ng():
                        pltpu.async_copy(x_h.at[src_v.at[ci + 1]],
                                         rows2_v.at[bn], gsems[bn])

                pltpu.make_async_copy(x_h.at[src_v.at[ci]],
                                      rows2_v.at[b], gsems[b]).wait()
                scale(rows2_v.at[b], ci)
                pltpu.async_copy(rows2_v.at[b], acc_sh.at[dst_v.at[ci]],
                                 ssems[b], add=True)
            return carry

        lax.fori_loop(0, _NCH // 2, pipe, 0)
        pltpu.make_async_copy(z_h.at[pl.ds(0, _CB)],
                              rows2_v.at[(_NCH - 1) % 2],
                              ssems[(_NCH - 1) % 2]).wait()
        plsc.subcore_barrier()
        pltpu.sync_copy(acc_sh.at[pl.ds(rbase, _RPT)],
                        out_h.at[cid, pl.ds(rbase, _RPT)])

        @pl.when(sid == 15)
        def _out_tail():
            pltpu.sync_copy(acc_sh.at[pl.ds(16 * _RPT, _N - 16 * _RPT)],
                            out_h.at[cid, pl.ds(16 * _RPT, _N - 16 * _RPT)])

    parts = sk(x, src3, dst3, w3, jnp.zeros((_N, k), jnp.float32))
    return parts[0] + parts[1]


def kernel(features, edge_src, edge_dst, edge_w, labels, mask,
           edge_pos_i, edge_pos_j, train_idx, mat01_intra, mat01_inter,
           W0, b0, W1, b1, Wh0, bh0, Wh1, bh1, Wc, bc):
    epad = _EP2 - _E
    src3 = jnp.pad(edge_src.astype(jnp.int32), (0, epad)).reshape(32, _NCH, _CB)
    dst3 = jnp.pad(edge_dst.astype(jnp.int32), (0, epad)).reshape(32, _NCH, _CB)
    w3 = jnp.pad(edge_w, (0, epad)).reshape(32, _NCH, _CB)
    sfeat = _spmm_sc(features, src3, dst3, w3, _D)
    h0 = jax.nn.relu(sfeat @ W0 + b0)
    hg0 = jax.nn.relu(sfeat @ Wh0 + bh0)
    s1cat = _spmm_sc(jnp.concatenate([h0, hg0], axis=1), src3, dst3, w3, _D)
    cv_gcn = _l2n(s1cat[:, :_H] @ W1 + b1)
    cv_hg = _l2n(s1cat[:, _H:] @ Wh1 + bh1)
    outputs = _l2n(0.6 * cv_gcn + 0.4 * cv_hg)

    m = mask / jnp.mean(mask)
    logp = jax.nn.log_softmax(outputs, axis=1)
    loss_q = jnp.mean(-(labels * logp).sum(axis=1) * m)

    a = Wc[:_C, 0]
    b = Wc[_C:, 0]
    u1 = jnp.pad(cv_gcn @ a + bc[0], (0, _NP - _N))
    v1 = jnp.pad(cv_hg @ b, (0, _NP - _N))
    u2 = jnp.pad(cv_hg @ a + bc[0], (0, _NP - _N))
    v2 = jnp.pad(cv_gcn @ b, (0, _NP - _N))
    epi = jnp.pad(edge_pos_i.astype(jnp.int32), (0, _EP - _E))
    epj = jnp.pad(edge_pos_j.astype(jnp.int32), (0, _EP - _E))
    s1e, s2e = _edge_logits_sc(u1, v1, u2, v2, epi, epj)
    p1 = -jnp.mean(jnp.log(jax.nn.sigmoid(s1e[:_E])))
    p2 = -jnp.mean(jnp.log(jax.nn.sigmoid(s2e[:_E])))
    p_e_xy = p1 + p2

    pad = _NP - _N
    g2p = jnp.pad(2.0 * cv_gcn, ((0, pad), (0, 0)))
    htp = jnp.pad(cv_hg.T, ((0, 0), (0, pad)))
    rowsum, colsum = _nxn_reductions(g2p, htp)
    d = jnp.exp(2.0 * jnp.sum(cv_gcn * cv_hg, axis=1))
    pn1 = d / (rowsum / _N)
    pn2 = d / (colsum / _N)
    closs = -0.9 * jnp.mean(jnp.log(jnp.concatenate([pn1, pn2], axis=0)))

    h1s = cv_gcn[train_idx]
    h2s = cv_hg[train_idx]
    padt = _TP - _T
    h1p2 = jnp.pad(2.0 * h1s, ((0, padt), (0, 0)))
    h2tp = jnp.pad(h2s.T, ((0, 0), (0, padt)))
    intrap = jnp.pad(mat01_intra, ((0, padt), (0, padt)))
    intratp = jnp.pad(mat01_intra.T, ((0, padt), (0, padt)))
    sprow, rowtot, spcol, coltot = _sup_reductions(h1p2, h2tp, intrap, intratp)
    rowsum01 = jnp.sum(mat01_intra, axis=1)
    sup1 = (sprow / rowsum01) / (rowtot / (_T - 1))
    sup2 = (spcol / rowsum01) / (coltot / (_T - 1))
    closs = closs - 0.9 * jnp.mean(jnp.log(jnp.concatenate([sup1, sup2], axis=0)))

    total = loss_q + 0.4 * p_e_xy + closs
    for v in (W0, b0, W1, b1, Wc, bc):
        total = total + _WD * 0.5 * jnp.sum(v ** 2)

    acc = jnp.mean((jnp.argmax(outputs, axis=1) ==
                    jnp.argmax(labels, axis=1)).astype(jnp.float32) * m)
    return (outputs, total, acc)


# speedup vs baseline: 1.8784x; 1.8099x over previous
import functools

import jax
import jax.numpy as jnp
from jax import lax
from jax.experimental import pallas as pl
from jax.experimental.pallas import tpu as pltpu
from jax.experimental.pallas import tpu_sc as plsc

_N = 10000
_E = 160000
_D = 128
_H = 64
_C = 16
_T = 1000
_WD = 5e-4

_TBR = 512
_TBC = 1024
_NP = 10240
_GI = _NP // _TBR
_GJ = _NP // _TBC

_TP = 1024
_RT = 128


def _l2n(x):
    n = jnp.sqrt(jnp.sum(x * x, axis=1, keepdims=True))
    return x / jnp.maximum(n, 1e-12)


def _nxn_body(g_ref, ht_ref, rowp_ref, colp_ref):
    s = jax.lax.dot_general(g_ref[...], ht_ref[...], (((1,), (0,)), ((), ())),
                            preferred_element_type=jnp.float32)
    p = jnp.exp(s)
    rowp_ref[...] = jnp.sum(p, axis=1, keepdims=True).reshape(1, _TBR, 1)
    colp_ref[...] = jnp.sum(p, axis=0, keepdims=True).reshape(1, 1, _TBC)


def _nxn_reductions(g2p, htp):
    rowp, colp = pl.pallas_call(
        _nxn_body,
        grid=(_GI, _GJ),
        in_specs=[
            pl.BlockSpec((_TBR, _C), lambda i, j: (i, 0)),
            pl.BlockSpec((_C, _TBC), lambda i, j: (0, j)),
        ],
        out_specs=[
            pl.BlockSpec((1, _TBR, 1), lambda i, j: (j, i, 0)),
            pl.BlockSpec((1, 1, _TBC), lambda i, j: (i, 0, j)),
        ],
        out_shape=[
            jax.ShapeDtypeStruct((_GJ, _NP, 1), jnp.float32),
            jax.ShapeDtypeStruct((_GI, 1, _NP), jnp.float32),
        ],
    )(g2p, htp)
    pad = _NP - _N
    rowsum = jnp.sum(rowp, axis=0)[:_N, 0] - pad
    colsum = jnp.sum(colp, axis=(0, 1))[:_N] - pad
    return rowsum, colsum


def _sup_body(h1_ref, h2t_ref, intra_ref, intrat_ref,
              sprow_ref, rowtot_ref, spcol_ref, coltot_ref):
    i = pl.program_id(0)
    s = jax.lax.dot_general(h1_ref[...], h2t_ref[...], (((1,), (0,)), ((), ())),
                            preferred_element_type=jnp.float32)
    p = jnp.exp(s)
    sprow_ref[...] = jnp.sum(p * intra_ref[...], axis=1, keepdims=True)
    rowtot_ref[...] = jnp.sum(p, axis=1, keepdims=True)

    @pl.when(i == 0)
    def _init():
        spcol_ref[...] = jnp.zeros_like(spcol_ref)
        coltot_ref[...] = jnp.zeros_like(coltot_ref)

    spcol_ref[...] += jnp.sum(p * intrat_ref[...], axis=0, keepdims=True)
    coltot_ref[...] += jnp.sum(p, axis=0, keepdims=True)


def _sup_reductions(h1p2, h2tp, intrap, intratp):
    grid = (_TP // _RT,)
    sprow, rowtot, spcol, coltot = pl.pallas_call(
        _sup_body,
        grid=grid,
        in_specs=[
            pl.BlockSpec((_RT, _C), lambda i: (i, 0)),
            pl.BlockSpec((_C, _TP), lambda i: (0, 0)),
            pl.BlockSpec((_RT, _TP), lambda i: (i, 0)),
            pl.BlockSpec((_RT, _TP), lambda i: (i, 0)),
        ],
        out_specs=[
            pl.BlockSpec((_RT, 1), lambda i: (i, 0)),
            pl.BlockSpec((_RT, 1), lambda i: (i, 0)),
            pl.BlockSpec((1, _TP), lambda i: (0, 0)),
            pl.BlockSpec((1, _TP), lambda i: (0, 0)),
        ],
        out_shape=[
            jax.ShapeDtypeStruct((_TP, 1), jnp.float32),
            jax.ShapeDtypeStruct((_TP, 1), jnp.float32),
            jax.ShapeDtypeStruct((1, _TP), jnp.float32),
            jax.ShapeDtypeStruct((1, _TP), jnp.float32),
        ],
    )(h1p2, h2tp, intrap, intratp)
    padt = _TP - _T
    return (sprow[:_T, 0], rowtot[:_T, 0] - padt,
            spcol[0, :_T], coltot[0, :_T] - padt)


_NW = 32
_EP = 160256
_EB = _EP // _NW


def _edge_logits_sc(u1, v1, u2, v2, epi, epj):
    mesh = plsc.VectorSubcoreMesh(core_axis_name="c", subcore_axis_name="s")

    @functools.partial(
        pl.kernel, mesh=mesh,
        out_type=[jax.ShapeDtypeStruct((_EP,), jnp.float32),
                  jax.ShapeDtypeStruct((_EP,), jnp.float32)],
        scratch_types=[pltpu.VMEM((_EB,), jnp.int32)] * 2
        + [pltpu.VMEM((_EB,), jnp.float32)] * 4
        + [pltpu.SemaphoreType.DMA],
    )
    def ek(u1_h, v1_h, u2_h, v2_h, epi_h, epj_h, s1_h, s2_h,
           ei_v, ej_v, a1_v, b1_v, a2_v, b2_v, sem):
        wid = lax.axis_index("s") * 2 + lax.axis_index("c")
        base = wid * _EB
        pltpu.sync_copy(epi_h.at[pl.ds(base, _EB)], ei_v)
        pltpu.sync_copy(epj_h.at[pl.ds(base, _EB)], ej_v)
        pltpu.async_copy(u1_h.at[ei_v], a1_v, sem)
        pltpu.async_copy(v1_h.at[ej_v], b1_v, sem)
        pltpu.async_copy(u2_h.at[ei_v], a2_v, sem)
        last = pltpu.async_copy(v2_h.at[ej_v], b2_v, sem)
        last.wait()
        last.wait()
        last.wait()
        last.wait()

        def body(k, carry):
            sl = pl.ds(k * 16, 16)
            a1_v[sl] = a1_v[sl] + b1_v[sl]
            a2_v[sl] = a2_v[sl] + b2_v[sl]
            return carry

        lax.fori_loop(0, _EB // 16, body, 0)
        pltpu.sync_copy(a1_v, s1_h.at[pl.ds(base, _EB)])
        pltpu.sync_copy(a2_v, s2_h.at[pl.ds(base, _EB)])

    return ek(u1, v1, u2, v2, epi, epj)


_NCH = 40
_CB = 128
_EP2 = 32 * _NCH * _CB
_RPT = 624


def _spmm_sc(x, src3, dst3, w3, k):
    mesh = plsc.VectorSubcoreMesh(core_axis_name="c", subcore_axis_name="s")

    @functools.partial(
        pl.kernel, mesh=mesh,
        out_type=jax.ShapeDtypeStruct((2, _N, k), jnp.float32),
        scratch_types=[
            pltpu.VMEM((_NCH, _CB), jnp.int32),
            pltpu.VMEM((_NCH, _CB), jnp.int32),
            pltpu.VMEM((_NCH, _CB), jnp.float32),
            pltpu.VMEM((2, _CB, k), jnp.float32),
            pltpu.VMEM_SHARED((_N, k), jnp.float32),
            pltpu.SemaphoreType.DMA,
            pltpu.SemaphoreType.DMA,
            pltpu.SemaphoreType.DMA,
            pltpu.SemaphoreType.DMA,
        ],
    )
    def sk(x_h, src_h, dst_h, w_h, z_h, out_h,
           src_v, dst_v, w_v, rows2_v, acc_sh,
           g0, g1, s0, s1):
        cid = lax.axis_index("c")
        sid = lax.axis_index("s")
        wid = cid * 16 + sid
        rbase = sid * _RPT
        pltpu.sync_copy(z_h.at[pl.ds(rbase, _RPT)],
                        acc_sh.at[pl.ds(rbase, _RPT)])

        @pl.when(sid == 15)
        def _zero_tail():
            pltpu.sync_copy(z_h.at[pl.ds(16 * _RPT, _N - 16 * _RPT)],
                            acc_sh.at[pl.ds(16 * _RPT, _N - 16 * _RPT)])
        pltpu.sync_copy(src_h.at[wid], src_v)
        pltpu.sync_copy(dst_h.at[wid], dst_v)
        pltpu.sync_copy(w_h.at[wid], w_v)
        plsc.subcore_barrier()

        gsems = (g0, g1)
        ssems = (s0, s1)

        def scale(rv, ci):
            def grp(g, c2):
                wv = w_v[ci, pl.ds(g * 16, 16)]
                for l in range(16):
                    wb = jnp.take(wv, jnp.full((16,), l, jnp.int32))
                    e = g * 16 + l
                    for kk in range(k // 16):
                        sl = pl.ds(kk * 16, 16)
                        rv[e, sl] = rv[e, sl] * wb
                return c2

            lax.fori_loop(0, _CB // 16, grp, 0)

        pltpu.async_copy(x_h.at[src_v.at[0]], rows2_v.at[0], gsems[0])

        def pipe(g, carry):
            for b in range(2):
                ci = 2 * g + b
                bn = 1 - b
                if b == 0:
                    @pl.when(g > 0)
                    def _dr0():
                        pltpu.make_async_copy(z_h.at[pl.ds(0, _CB)],
                                              rows2_v.at[bn], ssems[bn]).wait()

                    pltpu.async_copy(x_h.at[src_v.at[ci + 1]],
                                     rows2_v.at[bn], gsems[bn])
                else:
                    pltpu.make_async_copy(z_h.at[pl.ds(0, _CB)],
                                          rows2_v.at[bn], ssems[bn]).wait()

                    @pl.when(g < (_NCH // 2) - 1)
                    def _ng():
                        pltpu.async_copy(x_h.at[src_v.at[ci + 1]],
                                         rows2_v.at[bn], gsems[bn])

                pltpu.make_async_copy(x_h.at[src_v.at[ci]],
                                      rows2_v.at[b], gsems[b]).wait()
                scale(rows2_v.at[b], ci)
                pltpu.async_copy(rows2_v.at[b], acc_sh.at[dst_v.at[ci]],
                                 ssems[b], add=True)
            return carry

        lax.fori_loop(0, _NCH // 2, pipe, 0)
        pltpu.make_async_copy(z_h.at[pl.ds(0, _CB)],
                              rows2_v.at[(_NCH - 1) % 2],
                              ssems[(_NCH - 1) % 2]).wait()
        plsc.subcore_barrier()
        pltpu.sync_copy(acc_sh.at[pl.ds(rbase, _RPT)],
                        out_h.at[cid, pl.ds(rbase, _RPT)])

        @pl.when(sid == 15)
        def _out_tail():
            pltpu.sync_copy(acc_sh.at[pl.ds(16 * _RPT, _N - 16 * _RPT)],
                            out_h.at[cid, pl.ds(16 * _RPT, _N - 16 * _RPT)])

    parts = sk(x, src3, dst3, w3, jnp.zeros((_N, k), jnp.float32))
    return parts[0] + parts[1]


def kernel(features, edge_src, edge_dst, edge_w, labels, mask,
           edge_pos_i, edge_pos_j, train_idx, mat01_intra, mat01_inter,
           W0, b0, W1, b1, Wh0, bh0, Wh1, bh1, Wc, bc):
    epad = _EP2 - _E
    pad_idx = (jnp.arange(epad, dtype=jnp.int32) * 97) % _N
    src3 = jnp.concatenate([edge_src.astype(jnp.int32),
                            pad_idx]).reshape(32, _NCH, _CB)
    dst3 = jnp.concatenate([edge_dst.astype(jnp.int32),
                            pad_idx]).reshape(32, _NCH, _CB)
    w3 = jnp.pad(edge_w, (0, epad)).reshape(32, _NCH, _CB)
    sfeat = _spmm_sc(features, src3, dst3, w3, _D)
    h0 = jax.nn.relu(sfeat @ W0 + b0)
    hg0 = jax.nn.relu(sfeat @ Wh0 + bh0)
    s1cat = _spmm_sc(jnp.concatenate([h0, hg0], axis=1), src3, dst3, w3, _D)
    cv_gcn = _l2n(s1cat[:, :_H] @ W1 + b1)
    cv_hg = _l2n(s1cat[:, _H:] @ Wh1 + bh1)
    outputs = _l2n(0.6 * cv_gcn + 0.4 * cv_hg)

    m = mask / jnp.mean(mask)
    logp = jax.nn.log_softmax(outputs, axis=1)
    loss_q = jnp.mean(-(labels * logp).sum(axis=1) * m)

    a = Wc[:_C, 0]
    b = Wc[_C:, 0]
    u1 = jnp.pad(cv_gcn @ a + bc[0], (0, _NP - _N))
    v1 = jnp.pad(cv_hg @ b, (0, _NP - _N))
    u2 = jnp.pad(cv_hg @ a + bc[0], (0, _NP - _N))
    v2 = jnp.pad(cv_gcn @ b, (0, _NP - _N))
    epi = jnp.pad(edge_pos_i.astype(jnp.int32), (0, _EP - _E))
    epj = jnp.pad(edge_pos_j.astype(jnp.int32), (0, _EP - _E))
    s1e, s2e = _edge_logits_sc(u1, v1, u2, v2, epi, epj)
    p1 = -jnp.mean(jnp.log(jax.nn.sigmoid(s1e[:_E])))
    p2 = -jnp.mean(jnp.log(jax.nn.sigmoid(s2e[:_E])))
    p_e_xy = p1 + p2

    pad = _NP - _N
    g2p = jnp.pad(2.0 * cv_gcn, ((0, pad), (0, 0)))
    htp = jnp.pad(cv_hg.T, ((0, 0), (0, pad)))
    rowsum, colsum = _nxn_reductions(g2p, htp)
    d = jnp.exp(2.0 * jnp.sum(cv_gcn * cv_hg, axis=1))
    pn1 = d / (rowsum / _N)
    pn2 = d / (colsum / _N)
    closs = -0.9 * jnp.mean(jnp.log(jnp.concatenate([pn1, pn2], axis=0)))

    h1s = cv_gcn[train_idx]
    h2s = cv_hg[train_idx]
    padt = _TP - _T
    h1p2 = jnp.pad(2.0 * h1s, ((0, padt), (0, 0)))
    h2tp = jnp.pad(h2s.T, ((0, 0), (0, padt)))
    intrap = jnp.pad(mat01_intra, ((0, padt), (0, padt)))
    intratp = jnp.pad(mat01_intra.T, ((0, padt), (0, padt)))
    sprow, rowtot, spcol, coltot = _sup_reductions(h1p2, h2tp, intrap, intratp)
    rowsum01 = jnp.sum(mat01_intra, axis=1)
    sup1 = (sprow / rowsum01) / (rowtot / (_T - 1))
    sup2 = (spcol / rowsum01) / (coltot / (_T - 1))
    closs = closs - 0.9 * jnp.mean(jnp.log(jnp.concatenate([sup1, sup2], axis=0)))

    total = loss_q + 0.4 * p_e_xy + closs
    for v in (W0, b0, W1, b1, Wc, bc):
        total = total + _WD * 0.5 * jnp.sum(v ** 2)

    acc = jnp.mean((jnp.argmax(outputs, axis=1) ==
                    jnp.argmax(labels, axis=1)).astype(jnp.float32) * m)
    return (outputs, total, acc)


# speedup vs baseline: 1.9330x; 1.0291x over previous
import functools

import jax
import jax.numpy as jnp
from jax import lax
from jax.experimental import pallas as pl
from jax.experimental.pallas import tpu as pltpu
from jax.experimental.pallas import tpu_sc as plsc

_N = 10000
_E = 160000
_D = 128
_H = 64
_C = 16
_T = 1000
_WD = 5e-4

_TBR = 512
_TBC = 1024
_NP = 10240
_GI = _NP // _TBR
_GJ = _NP // _TBC

_TP = 1024
_RT = 128


def _l2n(x):
    n = jnp.sqrt(jnp.sum(x * x, axis=1, keepdims=True))
    return x / jnp.maximum(n, 1e-12)


def _nxn_body(g_ref, ht_ref, rowp_ref, colp_ref):
    s = jax.lax.dot_general(g_ref[...], ht_ref[...], (((1,), (0,)), ((), ())),
                            preferred_element_type=jnp.float32)
    p = jnp.exp(s)
    rowp_ref[...] = jnp.sum(p, axis=1, keepdims=True).reshape(1, _TBR, 1)
    colp_ref[...] = jnp.sum(p, axis=0, keepdims=True).reshape(1, 1, _TBC)


def _nxn_reductions(g2p, htp):
    rowp, colp = pl.pallas_call(
        _nxn_body,
        grid=(_GI, _GJ),
        in_specs=[
            pl.BlockSpec((_TBR, _C), lambda i, j: (i, 0)),
            pl.BlockSpec((_C, _TBC), lambda i, j: (0, j)),
        ],
        out_specs=[
            pl.BlockSpec((1, _TBR, 1), lambda i, j: (j, i, 0)),
            pl.BlockSpec((1, 1, _TBC), lambda i, j: (i, 0, j)),
        ],
        out_shape=[
            jax.ShapeDtypeStruct((_GJ, _NP, 1), jnp.float32),
            jax.ShapeDtypeStruct((_GI, 1, _NP), jnp.float32),
        ],
    )(g2p, htp)
    pad = _NP - _N
    rowsum = jnp.sum(rowp, axis=0)[:_N, 0] - pad
    colsum = jnp.sum(colp, axis=(0, 1))[:_N] - pad
    return rowsum, colsum


def _sup_body(h1_ref, h2t_ref, intra_ref, intrat_ref,
              sprow_ref, rowtot_ref, spcol_ref, coltot_ref):
    i = pl.program_id(0)
    s = jax.lax.dot_general(h1_ref[...], h2t_ref[...], (((1,), (0,)), ((), ())),
                            preferred_element_type=jnp.float32)
    p = jnp.exp(s)
    sprow_ref[...] = jnp.sum(p * intra_ref[...], axis=1, keepdims=True)
    rowtot_ref[...] = jnp.sum(p, axis=1, keepdims=True)

    @pl.when(i == 0)
    def _init():
        spcol_ref[...] = jnp.zeros_like(spcol_ref)
        coltot_ref[...] = jnp.zeros_like(coltot_ref)

    spcol_ref[...] += jnp.sum(p * intrat_ref[...], axis=0, keepdims=True)
    coltot_ref[...] += jnp.sum(p, axis=0, keepdims=True)


def _sup_reductions(h1p2, h2tp, intrap, intratp):
    grid = (_TP // _RT,)
    sprow, rowtot, spcol, coltot = pl.pallas_call(
        _sup_body,
        grid=grid,
        in_specs=[
            pl.BlockSpec((_RT, _C), lambda i: (i, 0)),
            pl.BlockSpec((_C, _TP), lambda i: (0, 0)),
            pl.BlockSpec((_RT, _TP), lambda i: (i, 0)),
            pl.BlockSpec((_RT, _TP), lambda i: (i, 0)),
        ],
        out_specs=[
            pl.BlockSpec((_RT, 1), lambda i: (i, 0)),
            pl.BlockSpec((_RT, 1), lambda i: (i, 0)),
            pl.BlockSpec((1, _TP), lambda i: (0, 0)),
            pl.BlockSpec((1, _TP), lambda i: (0, 0)),
        ],
        out_shape=[
            jax.ShapeDtypeStruct((_TP, 1), jnp.float32),
            jax.ShapeDtypeStruct((_TP, 1), jnp.float32),
            jax.ShapeDtypeStruct((1, _TP), jnp.float32),
            jax.ShapeDtypeStruct((1, _TP), jnp.float32),
        ],
    )(h1p2, h2tp, intrap, intratp)
    padt = _TP - _T
    return (sprow[:_T, 0], rowtot[:_T, 0] - padt,
            spcol[0, :_T], coltot[0, :_T] - padt)


_NW = 32
_EP = 160256
_EB = _EP // _NW


def _edge_logits_sc(u1, v1, u2, v2, epi, epj):
    mesh = plsc.VectorSubcoreMesh(core_axis_name="c", subcore_axis_name="s")

    @functools.partial(
        pl.kernel, mesh=mesh,
        out_type=[jax.ShapeDtypeStruct((_EP,), jnp.float32),
                  jax.ShapeDtypeStruct((_EP,), jnp.float32)],
        scratch_types=[pltpu.VMEM((_EB,), jnp.int32)] * 2
        + [pltpu.VMEM((_EB,), jnp.float32)] * 4
        + [pltpu.SemaphoreType.DMA],
    )
    def ek(u1_h, v1_h, u2_h, v2_h, epi_h, epj_h, s1_h, s2_h,
           ei_v, ej_v, a1_v, b1_v, a2_v, b2_v, sem):
        wid = lax.axis_index("s") * 2 + lax.axis_index("c")
        base = wid * _EB
        pltpu.sync_copy(epi_h.at[pl.ds(base, _EB)], ei_v)
        pltpu.sync_copy(epj_h.at[pl.ds(base, _EB)], ej_v)
        pltpu.async_copy(u1_h.at[ei_v], a1_v, sem)
        pltpu.async_copy(v1_h.at[ej_v], b1_v, sem)
        pltpu.async_copy(u2_h.at[ei_v], a2_v, sem)
        last = pltpu.async_copy(v2_h.at[ej_v], b2_v, sem)
        last.wait()
        last.wait()
        last.wait()
        last.wait()

        def body(k, carry):
            sl = pl.ds(k * 16, 16)
            a1_v[sl] = a1_v[sl] + b1_v[sl]
            a2_v[sl] = a2_v[sl] + b2_v[sl]
            return carry

        lax.fori_loop(0, _EB // 16, body, 0)
        pltpu.sync_copy(a1_v, s1_h.at[pl.ds(base, _EB)])
        pltpu.sync_copy(a2_v, s2_h.at[pl.ds(base, _EB)])

    return ek(u1, v1, u2, v2, epi, epj)


_NCH = 40
_CB = 128
_EP2 = 32 * _NCH * _CB
_RPT = 624


def _spmm_sc(x, src3, dst3, w3, k):
    mesh = plsc.VectorSubcoreMesh(core_axis_name="c", subcore_axis_name="s")

    @functools.partial(
        pl.kernel, mesh=mesh,
        out_type=jax.ShapeDtypeStruct((2, _N, k), jnp.float32),
        scratch_types=[
            pltpu.VMEM((_NCH, _CB), jnp.int32),
            pltpu.VMEM((_NCH, _CB), jnp.int32),
            pltpu.VMEM((_NCH, _CB), jnp.float32),
            pltpu.VMEM((2, _CB, k), jnp.float32),
            pltpu.VMEM_SHARED((_N, k), jnp.float32),
            pltpu.SemaphoreType.DMA,
            pltpu.SemaphoreType.DMA,
            pltpu.SemaphoreType.DMA,
            pltpu.SemaphoreType.DMA,
        ],
    )
    def sk(x_h, src_h, dst_h, w_h, z_h, out_h,
           src_v, dst_v, w_v, rows2_v, acc_sh,
           g0, g1, s0, s1):
        cid = lax.axis_index("c")
        sid = lax.axis_index("s")
        wid = cid * 16 + sid
        rbase = sid * _RPT
        pltpu.sync_copy(z_h.at[pl.ds(rbase, _RPT)],
                        acc_sh.at[pl.ds(rbase, _RPT)])

        @pl.when(sid == 15)
        def _zero_tail():
            pltpu.sync_copy(z_h.at[pl.ds(16 * _RPT, _N - 16 * _RPT)],
                            acc_sh.at[pl.ds(16 * _RPT, _N - 16 * _RPT)])
        pltpu.sync_copy(src_h.at[wid], src_v)
        pltpu.sync_copy(dst_h.at[wid], dst_v)
        pltpu.sync_copy(w_h.at[wid], w_v)
        plsc.subcore_barrier()

        gsems = (g0, g1)
        ssems = (s0, s1)

        def scale(rv, ci):
            def grp(g, c2):
                wv = w_v[ci, pl.ds(g * 16, 16)]
                for l in range(16):
                    wb = jnp.take(wv, jnp.full((16,), l, jnp.int32))
                    e = g * 16 + l
                    for kk in range(k // 16):
                        sl = pl.ds(kk * 16, 16)
                        rv[e, sl] = rv[e, sl] * wb
                return c2

            lax.fori_loop(0, _CB // 16, grp, 0)

        pltpu.async_copy(x_h.at[src_v.at[0]], rows2_v.at[0], gsems[0])

        def pipe(g, carry):
            for b in range(2):
                ci = 2 * g + b
                bn = 1 - b
                if b == 0:
                    @pl.when(g > 0)
                    def _dr0():
                        pltpu.make_async_copy(z_h.at[pl.ds(0, _CB)],
                                              rows2_v.at[bn], ssems[bn]).wait()

                    pltpu.async_copy(x_h.at[src_v.at[ci + 1]],
                                     rows2_v.at[bn], gsems[bn])
                else:
                    pltpu.make_async_copy(z_h.at[pl.ds(0, _CB)],
                                          rows2_v.at[bn], ssems[bn]).wait()

                    @pl.when(g < (_NCH // 2) - 1)
                    def _ng():
                        pltpu.async_copy(x_h.at[src_v.at[ci + 1]],
                                         rows2_v.at[bn], gsems[bn])

                pltpu.make_async_copy(x_h.at[src_v.at[ci]],
                                      rows2_v.at[b], gsems[b]).wait()
                scale(rows2_v.at[b], ci)
                pltpu.async_copy(rows2_v.at[b], acc_sh.at[dst_v.at[ci]],
                                 ssems[b], add=True)
            return carry

        lax.fori_loop(0, _NCH // 2, pipe, 0)
        pltpu.make_async_copy(z_h.at[pl.ds(0, _CB)],
                              rows2_v.at[(_NCH - 1) % 2],
                              ssems[(_NCH - 1) % 2]).wait()
        plsc.subcore_barrier()
        pltpu.sync_copy(acc_sh.at[pl.ds(rbase, _RPT)],
                        out_h.at[cid, pl.ds(rbase, _RPT)])

        @pl.when(sid == 15)
        def _out_tail():
            pltpu.sync_copy(acc_sh.at[pl.ds(16 * _RPT, _N - 16 * _RPT)],
                            out_h.at[cid, pl.ds(16 * _RPT, _N - 16 * _RPT)])

    parts = sk(x, src3, dst3, w3, jnp.zeros((_N, k), jnp.float32))
    return parts[0] + parts[1]


def kernel(features, edge_src, edge_dst, edge_w, labels, mask,
           edge_pos_i, edge_pos_j, train_idx, mat01_intra, mat01_inter,
           W0, b0, W1, b1, Wh0, bh0, Wh1, bh1, Wc, bc):
    epad = _EP2 - _E
    pad_idx = (jnp.arange(epad, dtype=jnp.int32) * 97) % _N
    src3 = jnp.concatenate([edge_src.astype(jnp.int32),
                            pad_idx]).reshape(32, _NCH, _CB)
    dst3 = jnp.concatenate([edge_dst.astype(jnp.int32),
                            pad_idx]).reshape(32, _NCH, _CB)
    w3 = jnp.pad(edge_w, (0, epad)).reshape(32, _NCH, _CB)
    sfeat = _spmm_sc(features, src3, dst3, w3, _D)
    h0 = jax.nn.relu(sfeat @ W0 + b0)
    hg0 = jax.nn.relu(sfeat @ Wh0 + bh0)
    s1cat = _spmm_sc(jnp.concatenate([h0, hg0], axis=1), src3, dst3, w3, _D)
    cv_gcn = _l2n(s1cat[:, :_H] @ W1 + b1)
    cv_hg = _l2n(s1cat[:, _H:] @ Wh1 + bh1)
    outputs = _l2n(0.6 * cv_gcn + 0.4 * cv_hg)

    m = mask / jnp.mean(mask)
    logp = jax.nn.log_softmax(outputs, axis=1)
    loss_q = jnp.mean(-(labels * logp).sum(axis=1) * m)

    a = Wc[:_C, 0]
    b = Wc[_C:, 0]
    u1 = jnp.pad(cv_gcn @ a + bc[0], (0, _NP - _N))
    v1 = jnp.pad(cv_hg @ b, (0, _NP - _N))
    u2 = jnp.pad(cv_hg @ a + bc[0], (0, _NP - _N))
    v2 = jnp.pad(cv_gcn @ b, (0, _NP - _N))
    epi = jnp.pad(edge_pos_i.astype(jnp.int32), (0, _EP - _E))
    epj = jnp.pad(edge_pos_j.astype(jnp.int32), (0, _EP - _E))
    s1e, s2e = _edge_logits_sc(u1, v1, u2, v2, epi, epj)
    p1 = -jnp.mean(jnp.log(jax.nn.sigmoid(s1e[:_E])))
    p2 = -jnp.mean(jnp.log(jax.nn.sigmoid(s2e[:_E])))
    p_e_xy = p1 + p2

    pad = _NP - _N
    g2p = jnp.pad(2.0 * cv_gcn, ((0, pad), (0, 0))).astype(jnp.bfloat16)
    htp = jnp.pad(cv_hg.T, ((0, 0), (0, pad))).astype(jnp.bfloat16)
    rowsum, colsum = _nxn_reductions(g2p, htp)
    d = jnp.exp(2.0 * jnp.sum(cv_gcn * cv_hg, axis=1))
    pn1 = d / (rowsum / _N)
    pn2 = d / (colsum / _N)
    closs = -0.9 * jnp.mean(jnp.log(jnp.concatenate([pn1, pn2], axis=0)))

    h1s = cv_gcn[train_idx]
    h2s = cv_hg[train_idx]
    padt = _TP - _T
    h1p2 = jnp.pad(2.0 * h1s, ((0, padt), (0, 0)))
    h2tp = jnp.pad(h2s.T, ((0, 0), (0, padt)))
    intrap = jnp.pad(mat01_intra, ((0, padt), (0, padt)))
    intratp = jnp.pad(mat01_intra.T, ((0, padt), (0, padt)))
    sprow, rowtot, spcol, coltot = _sup_reductions(h1p2, h2tp, intrap, intratp)
    rowsum01 = jnp.sum(mat01_intra, axis=1)
    sup1 = (sprow / rowsum01) / (rowtot / (_T - 1))
    sup2 = (spcol / rowsum01) / (coltot / (_T - 1))
    closs = closs - 0.9 * jnp.mean(jnp.log(jnp.concatenate([sup1, sup2], axis=0)))

    total = loss_q + 0.4 * p_e_xy + closs
    for v in (W0, b0, W1, b1, Wc, bc):
        total = total + _WD * 0.5 * jnp.sum(v ** 2)

    acc = jnp.mean((jnp.argmax(outputs, axis=1) ==
                    jnp.argmax(labels, axis=1)).astype(jnp.float32) * m)
    return (outputs, total, acc)


# speedup vs baseline: 1.9468x; 1.0071x over previous
import functools

import jax
import jax.numpy as jnp
from jax import lax
from jax.experimental import pallas as pl
from jax.experimental.pallas import tpu as pltpu
from jax.experimental.pallas import tpu_sc as plsc

_N = 10000
_E = 160000
_D = 128
_H = 64
_C = 16
_T = 1000
_WD = 5e-4

_TBR = 512
_TBC = 1024
_NP = 10240
_GI = _NP // _TBR
_GJ = _NP // _TBC

_TP = 1024
_RT = 128


def _l2n(x):
    n = jnp.sqrt(jnp.sum(x * x, axis=1, keepdims=True))
    return x / jnp.maximum(n, 1e-12)


def _nxn_body(g_ref, ht_ref, rowp_ref, colp_ref):
    s = jax.lax.dot_general(g_ref[...], ht_ref[...], (((1,), (0,)), ((), ())),
                            preferred_element_type=jnp.float32)
    p = jnp.exp(s)
    rowp_ref[...] = jnp.sum(p, axis=1, keepdims=True).reshape(1, _TBR, 1)
    colp_ref[...] = jnp.sum(p, axis=0, keepdims=True).reshape(1, 1, _TBC)


def _nxn_reductions(g2p, htp):
    rowp, colp = pl.pallas_call(
        _nxn_body,
        grid=(_GI, _GJ),
        in_specs=[
            pl.BlockSpec((_TBR, _C), lambda i, j: (i, 0)),
            pl.BlockSpec((_C, _TBC), lambda i, j: (0, j)),
        ],
        out_specs=[
            pl.BlockSpec((1, _TBR, 1), lambda i, j: (j, i, 0)),
            pl.BlockSpec((1, 1, _TBC), lambda i, j: (i, 0, j)),
        ],
        out_shape=[
            jax.ShapeDtypeStruct((_GJ, _NP, 1), jnp.float32),
            jax.ShapeDtypeStruct((_GI, 1, _NP), jnp.float32),
        ],
    )(g2p, htp)
    pad = _NP - _N
    rowsum = jnp.sum(rowp, axis=0)[:_N, 0] - pad
    colsum = jnp.sum(colp, axis=(0, 1))[:_N] - pad
    return rowsum, colsum


def _sup_body(h1_ref, h2t_ref, intra_ref, intrat_ref,
              sprow_ref, rowtot_ref, spcol_ref, coltot_ref):
    i = pl.program_id(0)
    s = jax.lax.dot_general(h1_ref[...], h2t_ref[...], (((1,), (0,)), ((), ())),
                            preferred_element_type=jnp.float32)
    p = jnp.exp(s)
    sprow_ref[...] = jnp.sum(p * intra_ref[...], axis=1, keepdims=True)
    rowtot_ref[...] = jnp.sum(p, axis=1, keepdims=True)

    @pl.when(i == 0)
    def _init():
        spcol_ref[...] = jnp.zeros_like(spcol_ref)
        coltot_ref[...] = jnp.zeros_like(coltot_ref)

    spcol_ref[...] += jnp.sum(p * intrat_ref[...], axis=0, keepdims=True)
    coltot_ref[...] += jnp.sum(p, axis=0, keepdims=True)


def _sup_reductions(h1p2, h2tp, intrap, intratp):
    grid = (_TP // _RT,)
    sprow, rowtot, spcol, coltot = pl.pallas_call(
        _sup_body,
        grid=grid,
        in_specs=[
            pl.BlockSpec((_RT, _C), lambda i: (i, 0)),
            pl.BlockSpec((_C, _TP), lambda i: (0, 0)),
            pl.BlockSpec((_RT, _TP), lambda i: (i, 0)),
            pl.BlockSpec((_RT, _TP), lambda i: (i, 0)),
        ],
        out_specs=[
            pl.BlockSpec((_RT, 1), lambda i: (i, 0)),
            pl.BlockSpec((_RT, 1), lambda i: (i, 0)),
            pl.BlockSpec((1, _TP), lambda i: (0, 0)),
            pl.BlockSpec((1, _TP), lambda i: (0, 0)),
        ],
        out_shape=[
            jax.ShapeDtypeStruct((_TP, 1), jnp.float32),
            jax.ShapeDtypeStruct((_TP, 1), jnp.float32),
            jax.ShapeDtypeStruct((1, _TP), jnp.float32),
            jax.ShapeDtypeStruct((1, _TP), jnp.float32),
        ],
    )(h1p2, h2tp, intrap, intratp)
    padt = _TP - _T
    return (sprow[:_T, 0], rowtot[:_T, 0] - padt,
            spcol[0, :_T], coltot[0, :_T] - padt)


_NW = 32
_EP = 160256
_EB = _EP // _NW


def _edge_logits_sc(u1, v1, u2, v2, epi, epj):
    mesh = plsc.VectorSubcoreMesh(core_axis_name="c", subcore_axis_name="s")

    @functools.partial(
        pl.kernel, mesh=mesh,
        out_type=[jax.ShapeDtypeStruct((_EP,), jnp.float32),
                  jax.ShapeDtypeStruct((_EP,), jnp.float32)],
        scratch_types=[pltpu.VMEM((_EB,), jnp.int32)] * 2
        + [pltpu.VMEM((_EB,), jnp.float32)] * 4
        + [pltpu.SemaphoreType.DMA],
    )
    def ek(u1_h, v1_h, u2_h, v2_h, epi_h, epj_h, s1_h, s2_h,
           ei_v, ej_v, a1_v, b1_v, a2_v, b2_v, sem):
        wid = lax.axis_index("s") * 2 + lax.axis_index("c")
        base = wid * _EB
        pltpu.sync_copy(epi_h.at[pl.ds(base, _EB)], ei_v)
        pltpu.sync_copy(epj_h.at[pl.ds(base, _EB)], ej_v)
        pltpu.async_copy(u1_h.at[ei_v], a1_v, sem)
        pltpu.async_copy(v1_h.at[ej_v], b1_v, sem)
        pltpu.async_copy(u2_h.at[ei_v], a2_v, sem)
        last = pltpu.async_copy(v2_h.at[ej_v], b2_v, sem)
        last.wait()
        last.wait()
        last.wait()
        last.wait()

        def body(k, carry):
            sl = pl.ds(k * 16, 16)
            a1_v[sl] = a1_v[sl] + b1_v[sl]
            a2_v[sl] = a2_v[sl] + b2_v[sl]
            return carry

        lax.fori_loop(0, _EB // 16, body, 0)
        pltpu.sync_copy(a1_v, s1_h.at[pl.ds(base, _EB)])
        pltpu.sync_copy(a2_v, s2_h.at[pl.ds(base, _EB)])

    return ek(u1, v1, u2, v2, epi, epj)


_NCH = 40
_CB = 128
_EP2 = 32 * _NCH * _CB
_RPT = 624


def _spmm_sc(x, src3, dst3, w3, k):
    mesh = plsc.VectorSubcoreMesh(core_axis_name="c", subcore_axis_name="s")

    @functools.partial(
        pl.kernel, mesh=mesh,
        out_type=jax.ShapeDtypeStruct((2, _N, k), jnp.float32),
        scratch_types=[
            pltpu.VMEM((_NCH, _CB), jnp.int32),
            pltpu.VMEM((_NCH, _CB), jnp.int32),
            pltpu.VMEM((_NCH, _CB), jnp.float32),
            pltpu.VMEM((2, _CB, k), jnp.float32),
            pltpu.VMEM_SHARED((_N, k), jnp.float32),
            pltpu.SemaphoreType.DMA,
            pltpu.SemaphoreType.DMA,
            pltpu.SemaphoreType.DMA,
            pltpu.SemaphoreType.DMA,
        ],
    )
    def sk(x_h, src_h, dst_h, w_h, z_h, out_h,
           src_v, dst_v, w_v, rows2_v, acc_sh,
           g0, g1, s0, s1):
        cid = lax.axis_index("c")
        sid = lax.axis_index("s")
        wid = cid * 16 + sid
        rbase = sid * _RPT
        pltpu.sync_copy(z_h.at[pl.ds(rbase, _RPT)],
                        acc_sh.at[pl.ds(rbase, _RPT)])

        @pl.when(sid == 15)
        def _zero_tail():
            pltpu.sync_copy(z_h.at[pl.ds(16 * _RPT, _N - 16 * _RPT)],
                            acc_sh.at[pl.ds(16 * _RPT, _N - 16 * _RPT)])
        pltpu.sync_copy(src_h.at[wid], src_v)
        pltpu.sync_copy(dst_h.at[wid], dst_v)
        pltpu.sync_copy(w_h.at[wid], w_v)
        plsc.subcore_barrier()

        gsems = (g0, g1)
        ssems = (s0, s1)

        def scale(rv, ci):
            def grp(g, c2):
                wv = w_v[ci, pl.ds(g * 16, 16)]
                for l in range(16):
                    wb = jnp.take(wv, jnp.full((16,), l, jnp.int32))
                    e = g * 16 + l
                    for kk in range(k // 16):
                        sl = pl.ds(kk * 16, 16)
                        rv[e, sl] = rv[e, sl] * wb
                return c2

            lax.fori_loop(0, _CB // 16, grp, 0)

        pltpu.async_copy(x_h.at[src_v.at[0]], rows2_v.at[0], gsems[0])

        def pipe(g, carry):
            for b in range(2):
                ci = 2 * g + b
                bn = 1 - b
                if b == 0:
                    @pl.when(g > 0)
                    def _dr0():
                        pltpu.make_async_copy(z_h.at[pl.ds(0, _CB)],
                                              rows2_v.at[bn], ssems[bn]).wait()

                    pltpu.async_copy(x_h.at[src_v.at[ci + 1]],
                                     rows2_v.at[bn], gsems[bn])
                else:
                    pltpu.make_async_copy(z_h.at[pl.ds(0, _CB)],
                                          rows2_v.at[bn], ssems[bn]).wait()

                    @pl.when(g < (_NCH // 2) - 1)
                    def _ng():
                        pltpu.async_copy(x_h.at[src_v.at[ci + 1]],
                                         rows2_v.at[bn], gsems[bn])

                pltpu.make_async_copy(x_h.at[src_v.at[ci]],
                                      rows2_v.at[b], gsems[b]).wait()
                scale(rows2_v.at[b], ci)
                pltpu.async_copy(rows2_v.at[b], acc_sh.at[dst_v.at[ci]],
                                 ssems[b], add=True)
            return carry

        lax.fori_loop(0, _NCH // 2, pipe, 0)
        pltpu.make_async_copy(z_h.at[pl.ds(0, _CB)],
                              rows2_v.at[(_NCH - 1) % 2],
                              ssems[(_NCH - 1) % 2]).wait()
        plsc.subcore_barrier()
        pltpu.sync_copy(acc_sh.at[pl.ds(rbase, _RPT)],
                        out_h.at[cid, pl.ds(rbase, _RPT)])

        @pl.when(sid == 15)
        def _out_tail():
            pltpu.sync_copy(acc_sh.at[pl.ds(16 * _RPT, _N - 16 * _RPT)],
                            out_h.at[cid, pl.ds(16 * _RPT, _N - 16 * _RPT)])

    parts = sk(x, src3, dst3, w3, jnp.zeros((_N, k), jnp.float32))
    return parts[0] + parts[1]


def kernel(features, edge_src, edge_dst, edge_w, labels, mask,
           edge_pos_i, edge_pos_j, train_idx, mat01_intra, mat01_inter,
           W0, b0, W1, b1, Wh0, bh0, Wh1, bh1, Wc, bc):
    epad = _EP2 - _E
    pad_idx = (jnp.arange(epad, dtype=jnp.int32) * 97) % _N
    src3 = jnp.concatenate([edge_src.astype(jnp.int32),
                            pad_idx]).reshape(32, _NCH, _CB)
    dst3 = jnp.concatenate([edge_dst.astype(jnp.int32),
                            pad_idx]).reshape(32, _NCH, _CB)
    w3 = jnp.pad(edge_w, (0, epad)).reshape(32, _NCH, _CB)
    sfeat = _spmm_sc(features, src3, dst3, w3, _D)
    h0cat = jax.nn.relu(sfeat @ jnp.concatenate([W0, Wh0], axis=1)
                        + jnp.concatenate([b0, bh0]))
    s1cat = _spmm_sc(h0cat, src3, dst3, w3, _D)
    cv_gcn = _l2n(s1cat[:, :_H] @ W1 + b1)
    cv_hg = _l2n(s1cat[:, _H:] @ Wh1 + bh1)
    outputs = _l2n(0.6 * cv_gcn + 0.4 * cv_hg)

    m = mask / jnp.mean(mask)
    logp = jax.nn.log_softmax(outputs, axis=1)
    loss_q = jnp.mean(-(labels * logp).sum(axis=1) * m)

    a = Wc[:_C, 0]
    b = Wc[_C:, 0]
    u1 = jnp.pad(cv_gcn @ a + bc[0], (0, _NP - _N))
    v1 = jnp.pad(cv_hg @ b, (0, _NP - _N))
    u2 = jnp.pad(cv_hg @ a + bc[0], (0, _NP - _N))
    v2 = jnp.pad(cv_gcn @ b, (0, _NP - _N))
    epi = jnp.pad(edge_pos_i.astype(jnp.int32), (0, _EP - _E))
    epj = jnp.pad(edge_pos_j.astype(jnp.int32), (0, _EP - _E))
    s1e, s2e = _edge_logits_sc(u1, v1, u2, v2, epi, epj)
    p1 = -jnp.mean(jnp.log(jax.nn.sigmoid(s1e[:_E])))
    p2 = -jnp.mean(jnp.log(jax.nn.sigmoid(s2e[:_E])))
    p_e_xy = p1 + p2

    pad = _NP - _N
    g2p = jnp.pad(2.0 * cv_gcn, ((0, pad), (0, 0))).astype(jnp.bfloat16)
    htp = jnp.pad(cv_hg.T, ((0, 0), (0, pad))).astype(jnp.bfloat16)
    rowsum, colsum = _nxn_reductions(g2p, htp)
    d = jnp.exp(2.0 * jnp.sum(cv_gcn * cv_hg, axis=1))
    pn1 = d / (rowsum / _N)
    pn2 = d / (colsum / _N)
    closs = -0.9 * jnp.mean(jnp.log(jnp.concatenate([pn1, pn2], axis=0)))

    h1s = cv_gcn[train_idx]
    h2s = cv_hg[train_idx]
    padt = _TP - _T
    h1p2 = jnp.pad(2.0 * h1s, ((0, padt), (0, 0)))
    h2tp = jnp.pad(h2s.T, ((0, 0), (0, padt)))
    intrap = jnp.pad(mat01_intra, ((0, padt), (0, padt)))
    intratp = jnp.pad(mat01_intra.T, ((0, padt), (0, padt)))
    sprow, rowtot, spcol, coltot = _sup_reductions(h1p2, h2tp, intrap, intratp)
    rowsum01 = jnp.sum(mat01_intra, axis=1)
    sup1 = (sprow / rowsum01) / (rowtot / (_T - 1))
    sup2 = (spcol / rowsum01) / (coltot / (_T - 1))
    closs = closs - 0.9 * jnp.mean(jnp.log(jnp.concatenate([sup1, sup2], axis=0)))

    total = loss_q + 0.4 * p_e_xy + closs
    for v in (W0, b0, W1, b1, Wc, bc):
        total = total + _WD * 0.5 * jnp.sum(v ** 2)

    acc = jnp.mean((jnp.argmax(outputs, axis=1) ==
                    jnp.argmax(labels, axis=1)).astype(jnp.float32) * m)
    return (outputs, total, acc)


# speedup vs baseline: 2.2210x; 1.1409x over previous
import functools

import jax
import jax.numpy as jnp
from jax import lax
from jax.experimental import pallas as pl
from jax.experimental.pallas import tpu as pltpu
from jax.experimental.pallas import tpu_sc as plsc

_N = 10000
_E = 160000
_D = 128
_H = 64
_C = 16
_T = 1000
_WD = 5e-4

_TBR = 1024
_TBC = 1024
_NP = 10240
_GI = _NP // _TBR
_GJ = _NP // _TBC

_TP = 1024
_RT = 128


def _l2n(x):
    n = jnp.sqrt(jnp.sum(x * x, axis=1, keepdims=True))
    return x / jnp.maximum(n, 1e-12)


def _nxn_body(g_ref, ht_ref, rowp_ref, colp_ref):
    s = jax.lax.dot_general(g_ref[...], ht_ref[...], (((1,), (0,)), ((), ())),
                            preferred_element_type=jnp.float32)
    p = jnp.exp(s)
    rowp_ref[...] = jnp.sum(p, axis=1, keepdims=True).reshape(1, _TBR, 1)
    colp_ref[...] = jnp.sum(p, axis=0, keepdims=True).reshape(1, 1, _TBC)


def _nxn_reductions(g2p, htp):
    rowp, colp = pl.pallas_call(
        _nxn_body,
        grid=(_GI, _GJ),
        in_specs=[
            pl.BlockSpec((_TBR, _C), lambda i, j: (i, 0)),
            pl.BlockSpec((_C, _TBC), lambda i, j: (0, j)),
        ],
        out_specs=[
            pl.BlockSpec((1, _TBR, 1), lambda i, j: (j, i, 0)),
            pl.BlockSpec((1, 1, _TBC), lambda i, j: (i, 0, j)),
        ],
        out_shape=[
            jax.ShapeDtypeStruct((_GJ, _NP, 1), jnp.float32),
            jax.ShapeDtypeStruct((_GI, 1, _NP), jnp.float32),
        ],
    )(g2p, htp)
    pad = _NP - _N
    rowsum = jnp.sum(rowp, axis=0)[:_N, 0] - pad
    colsum = jnp.sum(colp, axis=(0, 1))[:_N] - pad
    return rowsum, colsum


def _sup_body(h1_ref, h2t_ref, intra_ref, intrat_ref,
              sprow_ref, rowtot_ref, spcol_ref, coltot_ref):
    i = pl.program_id(0)
    s = jax.lax.dot_general(h1_ref[...], h2t_ref[...], (((1,), (0,)), ((), ())),
                            preferred_element_type=jnp.float32)
    p = jnp.exp(s)
    sprow_ref[...] = jnp.sum(p * intra_ref[...], axis=1, keepdims=True)
    rowtot_ref[...] = jnp.sum(p, axis=1, keepdims=True)

    @pl.when(i == 0)
    def _init():
        spcol_ref[...] = jnp.zeros_like(spcol_ref)
        coltot_ref[...] = jnp.zeros_like(coltot_ref)

    spcol_ref[...] += jnp.sum(p * intrat_ref[...], axis=0, keepdims=True)
    coltot_ref[...] += jnp.sum(p, axis=0, keepdims=True)


def _sup_reductions(h1p2, h2tp, intrap, intratp):
    grid = (_TP // _RT,)
    sprow, rowtot, spcol, coltot = pl.pallas_call(
        _sup_body,
        grid=grid,
        in_specs=[
            pl.BlockSpec((_RT, _C), lambda i: (i, 0)),
            pl.BlockSpec((_C, _TP), lambda i: (0, 0)),
            pl.BlockSpec((_RT, _TP), lambda i: (i, 0)),
            pl.BlockSpec((_RT, _TP), lambda i: (i, 0)),
        ],
        out_specs=[
            pl.BlockSpec((_RT, 1), lambda i: (i, 0)),
            pl.BlockSpec((_RT, 1), lambda i: (i, 0)),
            pl.BlockSpec((1, _TP), lambda i: (0, 0)),
            pl.BlockSpec((1, _TP), lambda i: (0, 0)),
        ],
        out_shape=[
            jax.ShapeDtypeStruct((_TP, 1), jnp.float32),
            jax.ShapeDtypeStruct((_TP, 1), jnp.float32),
            jax.ShapeDtypeStruct((1, _TP), jnp.float32),
            jax.ShapeDtypeStruct((1, _TP), jnp.float32),
        ],
    )(h1p2, h2tp, intrap, intratp)
    padt = _TP - _T
    return (sprow[:_T, 0], rowtot[:_T, 0] - padt,
            spcol[0, :_T], coltot[0, :_T] - padt)


_NW = 32
_EP = 160256
_EB = _EP // _NW


def _edge_logits_sc(u1, v1, u2, v2, epi, epj):
    mesh = plsc.VectorSubcoreMesh(core_axis_name="c", subcore_axis_name="s")

    @functools.partial(
        pl.kernel, mesh=mesh,
        out_type=[jax.ShapeDtypeStruct((_EP,), jnp.float32),
                  jax.ShapeDtypeStruct((_EP,), jnp.float32)],
        scratch_types=[pltpu.VMEM((_EB,), jnp.int32)] * 2
        + [pltpu.VMEM((_EB,), jnp.float32)] * 4
        + [pltpu.SemaphoreType.DMA],
    )
    def ek(u1_h, v1_h, u2_h, v2_h, epi_h, epj_h, s1_h, s2_h,
           ei_v, ej_v, a1_v, b1_v, a2_v, b2_v, sem):
        wid = lax.axis_index("s") * 2 + lax.axis_index("c")
        base = wid * _EB
        pltpu.sync_copy(epi_h.at[pl.ds(base, _EB)], ei_v)
        pltpu.sync_copy(epj_h.at[pl.ds(base, _EB)], ej_v)
        pltpu.async_copy(u1_h.at[ei_v], a1_v, sem)
        pltpu.async_copy(v1_h.at[ej_v], b1_v, sem)
        pltpu.async_copy(u2_h.at[ei_v], a2_v, sem)
        last = pltpu.async_copy(v2_h.at[ej_v], b2_v, sem)
        last.wait()
        last.wait()
        last.wait()
        last.wait()

        def body(k, carry):
            sl = pl.ds(k * 16, 16)
            a1_v[sl] = a1_v[sl] + b1_v[sl]
            a2_v[sl] = a2_v[sl] + b2_v[sl]
            return carry

        lax.fori_loop(0, _EB // 16, body, 0)
        pltpu.sync_copy(a1_v, s1_h.at[pl.ds(base, _EB)])
        pltpu.sync_copy(a2_v, s2_h.at[pl.ds(base, _EB)])

    return ek(u1, v1, u2, v2, epi, epj)


_NCH = 40
_CB = 128
_EP2 = 32 * _NCH * _CB
_RPT = 624


def _spmm_sc(x, src3, dst3, w3, k):
    mesh = plsc.VectorSubcoreMesh(core_axis_name="c", subcore_axis_name="s")

    @functools.partial(
        pl.kernel, mesh=mesh,
        out_type=jax.ShapeDtypeStruct((2, _N, k), jnp.float32),
        scratch_types=[
            pltpu.VMEM((_NCH, _CB), jnp.int32),
            pltpu.VMEM((_NCH, _CB), jnp.int32),
            pltpu.VMEM((_NCH, _CB), jnp.float32),
            pltpu.VMEM((2, _CB, k), jnp.float32),
            pltpu.VMEM_SHARED((_N, k), jnp.float32),
            pltpu.SemaphoreType.DMA,
            pltpu.SemaphoreType.DMA,
            pltpu.SemaphoreType.DMA,
            pltpu.SemaphoreType.DMA,
        ],
    )
    def sk(x_h, src_h, dst_h, w_h, z_h, out_h,
           src_v, dst_v, w_v, rows2_v, acc_sh,
           g0, g1, s0, s1):
        cid = lax.axis_index("c")
        sid = lax.axis_index("s")
        wid = cid * 16 + sid
        rbase = sid * _RPT
        pltpu.sync_copy(z_h.at[pl.ds(rbase, _RPT)],
                        acc_sh.at[pl.ds(rbase, _RPT)])

        @pl.when(sid == 15)
        def _zero_tail():
            pltpu.sync_copy(z_h.at[pl.ds(16 * _RPT, _N - 16 * _RPT)],
                            acc_sh.at[pl.ds(16 * _RPT, _N - 16 * _RPT)])
        pltpu.sync_copy(src_h.at[wid], src_v)
        pltpu.sync_copy(dst_h.at[wid], dst_v)
        pltpu.sync_copy(w_h.at[wid], w_v)
        plsc.subcore_barrier()

        gsems = (g0, g1)
        ssems = (s0, s1)

        def scale(rv, ci):
            def grp(g, c2):
                wv = w_v[ci, pl.ds(g * 16, 16)]
                for l in range(16):
                    wb = jnp.take(wv, jnp.full((16,), l, jnp.int32))
                    e = g * 16 + l
                    for kk in range(k // 16):
                        sl = pl.ds(kk * 16, 16)
                        rv[e, sl] = rv[e, sl] * wb
                return c2

            lax.fori_loop(0, _CB // 16, grp, 0)

        pltpu.async_copy(x_h.at[src_v.at[0]], rows2_v.at[0], gsems[0])

        def pipe(g, carry):
            for b in range(2):
                ci = 2 * g + b
                bn = 1 - b
                if b == 0:
                    @pl.when(g > 0)
                    def _dr0():
                        pltpu.make_async_copy(z_h.at[pl.ds(0, _CB)],
                                              rows2_v.at[bn], ssems[bn]).wait()

                    pltpu.async_copy(x_h.at[src_v.at[ci + 1]],
                                     rows2_v.at[bn], gsems[bn])
                else:
                    pltpu.make_async_copy(z_h.at[pl.ds(0, _CB)],
                                          rows2_v.at[bn], ssems[bn]).wait()

                    @pl.when(g < (_NCH // 2) - 1)
                    def _ng():
                        pltpu.async_copy(x_h.at[src_v.at[ci + 1]],
                                         rows2_v.at[bn], gsems[bn])

                pltpu.make_async_copy(x_h.at[src_v.at[ci]],
                                      rows2_v.at[b], gsems[b]).wait()
                scale(rows2_v.at[b], ci)
                pltpu.async_copy(rows2_v.at[b], acc_sh.at[dst_v.at[ci]],
                                 ssems[b], add=True)
            return carry

        lax.fori_loop(0, _NCH // 2, pipe, 0)
        pltpu.make_async_copy(z_h.at[pl.ds(0, _CB)],
                              rows2_v.at[(_NCH - 1) % 2],
                              ssems[(_NCH - 1) % 2]).wait()
        plsc.subcore_barrier()
        pltpu.sync_copy(acc_sh.at[pl.ds(rbase, _RPT)],
                        out_h.at[cid, pl.ds(rbase, _RPT)])

        @pl.when(sid == 15)
        def _out_tail():
            pltpu.sync_copy(acc_sh.at[pl.ds(16 * _RPT, _N - 16 * _RPT)],
                            out_h.at[cid, pl.ds(16 * _RPT, _N - 16 * _RPT)])

    parts = sk(x, src3, dst3, w3, jnp.zeros((_N, k), jnp.float32))
    return parts[0] + parts[1]


def kernel(features, edge_src, edge_dst, edge_w, labels, mask,
           edge_pos_i, edge_pos_j, train_idx, mat01_intra, mat01_inter,
           W0, b0, W1, b1, Wh0, bh0, Wh1, bh1, Wc, bc):
    epad = _EP2 - _E
    pad_idx = (jnp.arange(epad, dtype=jnp.int32) * 97) % _N
    src3 = jnp.concatenate([edge_src.astype(jnp.int32),
                            pad_idx]).reshape(32, _NCH, _CB)
    dst3 = jnp.concatenate([edge_dst.astype(jnp.int32),
                            pad_idx]).reshape(32, _NCH, _CB)
    w3 = jnp.pad(edge_w, (0, epad)).reshape(32, _NCH, _CB)
    sfeat = _spmm_sc(features, src3, dst3, w3, _D)
    h0cat = jax.nn.relu(sfeat @ jnp.concatenate([W0, Wh0], axis=1)
                        + jnp.concatenate([b0, bh0]))
    s1cat = _spmm_sc(h0cat, src3, dst3, w3, _D)
    cv_gcn = _l2n(s1cat[:, :_H] @ W1 + b1)
    cv_hg = _l2n(s1cat[:, _H:] @ Wh1 + bh1)
    outputs = _l2n(0.6 * cv_gcn + 0.4 * cv_hg)

    m = mask / jnp.mean(mask)
    logp = jax.nn.log_softmax(outputs, axis=1)
    loss_q = jnp.mean(-(labels * logp).sum(axis=1) * m)

    a = Wc[:_C, 0]
    b = Wc[_C:, 0]
    u1 = jnp.pad(cv_gcn @ a + bc[0], (0, _NP - _N))
    v1 = jnp.pad(cv_hg @ b, (0, _NP - _N))
    u2 = jnp.pad(cv_hg @ a + bc[0], (0, _NP - _N))
    v2 = jnp.pad(cv_gcn @ b, (0, _NP - _N))
    epi = jnp.pad(edge_pos_i.astype(jnp.int32), (0, _EP - _E))
    epj = jnp.pad(edge_pos_j.astype(jnp.int32), (0, _EP - _E))
    s1e, s2e = _edge_logits_sc(u1, v1, u2, v2, epi, epj)
    p1 = -jnp.mean(jnp.log(jax.nn.sigmoid(s1e[:_E])))
    p2 = -jnp.mean(jnp.log(jax.nn.sigmoid(s2e[:_E])))
    p_e_xy = p1 + p2

    pad = _NP - _N
    g2p = jnp.pad(2.0 * cv_gcn, ((0, pad), (0, 0))).astype(jnp.bfloat16)
    htp = jnp.pad(cv_hg.T, ((0, 0), (0, pad))).astype(jnp.bfloat16)
    rowsum, colsum = _nxn_reductions(g2p, htp)
    d = jnp.exp(2.0 * jnp.sum(cv_gcn * cv_hg, axis=1))
    pn1 = d / (rowsum / _N)
    pn2 = d / (colsum / _N)
    closs = -0.9 * jnp.mean(jnp.log(jnp.concatenate([pn1, pn2], axis=0)))

    h1s = cv_gcn[train_idx]
    h2s = cv_hg[train_idx]
    padt = _TP - _T
    h1p2 = jnp.pad(2.0 * h1s, ((0, padt), (0, 0)))
    h2tp = jnp.pad(h2s.T, ((0, 0), (0, padt)))
    intrap = jnp.pad(mat01_intra, ((0, padt), (0, padt)))
    intratp = jnp.pad(mat01_intra.T, ((0, padt), (0, padt)))
    sprow, rowtot, spcol, coltot = _sup_reductions(h1p2, h2tp, intrap, intratp)
    rowsum01 = jnp.sum(mat01_intra, axis=1)
    sup1 = (sprow / rowsum01) / (rowtot / (_T - 1))
    sup2 = (spcol / rowsum01) / (coltot / (_T - 1))
    closs = closs - 0.9 * jnp.mean(jnp.log(jnp.concatenate([sup1, sup2], axis=0)))

    total = loss_q + 0.4 * p_e_xy + closs
    for v in (W0, b0, W1, b1, Wc, bc):
        total = total + _WD * 0.5 * jnp.sum(v ** 2)

    acc = jnp.mean((jnp.argmax(outputs, axis=1) ==
                    jnp.argmax(labels, axis=1)).astype(jnp.float32) * m)
    return (outputs, total, acc)


# speedup vs baseline: 2.3900x; 1.0761x over previous
import functools

import jax
import jax.numpy as jnp
from jax import lax
from jax.experimental import pallas as pl
from jax.experimental.pallas import tpu as pltpu
from jax.experimental.pallas import tpu_sc as plsc

_N = 10000
_E = 160000
_D = 128
_H = 64
_C = 16
_T = 1000
_WD = 5e-4

_TBR = 1024
_TBC = 2048
_NP = 10240
_GI = _NP // _TBR
_GJ = _NP // _TBC

_TP = 1024
_RT = 128


def _l2n(x):
    n = jnp.sqrt(jnp.sum(x * x, axis=1, keepdims=True))
    return x / jnp.maximum(n, 1e-12)


def _nxn_body(g_ref, ht_ref, rowp_ref, colp_ref):
    s = jax.lax.dot_general(g_ref[...], ht_ref[...], (((1,), (0,)), ((), ())),
                            preferred_element_type=jnp.float32)
    p = jnp.exp(s)
    rowp_ref[...] = jnp.sum(p, axis=1, keepdims=True).reshape(1, _TBR, 1)
    colp_ref[...] = jnp.sum(p, axis=0, keepdims=True).reshape(1, 1, _TBC)


def _nxn_reductions(g2p, htp):
    rowp, colp = pl.pallas_call(
        _nxn_body,
        grid=(_GI, _GJ),
        in_specs=[
            pl.BlockSpec((_TBR, _C), lambda i, j: (i, 0)),
            pl.BlockSpec((_C, _TBC), lambda i, j: (0, j)),
        ],
        out_specs=[
            pl.BlockSpec((1, _TBR, 1), lambda i, j: (j, i, 0)),
            pl.BlockSpec((1, 1, _TBC), lambda i, j: (i, 0, j)),
        ],
        out_shape=[
            jax.ShapeDtypeStruct((_GJ, _NP, 1), jnp.float32),
            jax.ShapeDtypeStruct((_GI, 1, _NP), jnp.float32),
        ],
    )(g2p, htp)
    pad = _NP - _N
    rowsum = jnp.sum(rowp, axis=0)[:_N, 0] - pad
    colsum = jnp.sum(colp, axis=(0, 1))[:_N] - pad
    return rowsum, colsum


def _sup_body(h1_ref, h2t_ref, intra_ref, intrat_ref,
              sprow_ref, rowtot_ref, spcol_ref, coltot_ref):
    i = pl.program_id(0)
    s = jax.lax.dot_general(h1_ref[...], h2t_ref[...], (((1,), (0,)), ((), ())),
                            preferred_element_type=jnp.float32)
    p = jnp.exp(s)
    sprow_ref[...] = jnp.sum(p * intra_ref[...], axis=1, keepdims=True)
    rowtot_ref[...] = jnp.sum(p, axis=1, keepdims=True)

    @pl.when(i == 0)
    def _init():
        spcol_ref[...] = jnp.zeros_like(spcol_ref)
        coltot_ref[...] = jnp.zeros_like(coltot_ref)

    spcol_ref[...] += jnp.sum(p * intrat_ref[...], axis=0, keepdims=True)
    coltot_ref[...] += jnp.sum(p, axis=0, keepdims=True)


def _sup_reductions(h1p2, h2tp, intrap, intratp):
    grid = (_TP // _RT,)
    sprow, rowtot, spcol, coltot = pl.pallas_call(
        _sup_body,
        grid=grid,
        in_specs=[
            pl.BlockSpec((_RT, _C), lambda i: (i, 0)),
            pl.BlockSpec((_C, _TP), lambda i: (0, 0)),
            pl.BlockSpec((_RT, _TP), lambda i: (i, 0)),
            pl.BlockSpec((_RT, _TP), lambda i: (i, 0)),
        ],
        out_specs=[
            pl.BlockSpec((_RT, 1), lambda i: (i, 0)),
            pl.BlockSpec((_RT, 1), lambda i: (i, 0)),
            pl.BlockSpec((1, _TP), lambda i: (0, 0)),
            pl.BlockSpec((1, _TP), lambda i: (0, 0)),
        ],
        out_shape=[
            jax.ShapeDtypeStruct((_TP, 1), jnp.float32),
            jax.ShapeDtypeStruct((_TP, 1), jnp.float32),
            jax.ShapeDtypeStruct((1, _TP), jnp.float32),
            jax.ShapeDtypeStruct((1, _TP), jnp.float32),
        ],
    )(h1p2, h2tp, intrap, intratp)
    padt = _TP - _T
    return (sprow[:_T, 0], rowtot[:_T, 0] - padt,
            spcol[0, :_T], coltot[0, :_T] - padt)


_NW = 32
_EP = 160256
_EB = _EP // _NW


def _edge_logits_sc(u1, v1, u2, v2, epi, epj):
    mesh = plsc.VectorSubcoreMesh(core_axis_name="c", subcore_axis_name="s")

    @functools.partial(
        pl.kernel, mesh=mesh,
        out_type=[jax.ShapeDtypeStruct((_EP,), jnp.float32),
                  jax.ShapeDtypeStruct((_EP,), jnp.float32)],
        scratch_types=[pltpu.VMEM((_EB,), jnp.int32)] * 2
        + [pltpu.VMEM((_EB,), jnp.float32)] * 4
        + [pltpu.SemaphoreType.DMA],
    )
    def ek(u1_h, v1_h, u2_h, v2_h, epi_h, epj_h, s1_h, s2_h,
           ei_v, ej_v, a1_v, b1_v, a2_v, b2_v, sem):
        wid = lax.axis_index("s") * 2 + lax.axis_index("c")
        base = wid * _EB
        pltpu.sync_copy(epi_h.at[pl.ds(base, _EB)], ei_v)
        pltpu.sync_copy(epj_h.at[pl.ds(base, _EB)], ej_v)
        pltpu.async_copy(u1_h.at[ei_v], a1_v, sem)
        pltpu.async_copy(v1_h.at[ej_v], b1_v, sem)
        pltpu.async_copy(u2_h.at[ei_v], a2_v, sem)
        last = pltpu.async_copy(v2_h.at[ej_v], b2_v, sem)
        last.wait()
        last.wait()
        last.wait()
        last.wait()

        def body(k, carry):
            sl = pl.ds(k * 16, 16)
            a1_v[sl] = a1_v[sl] + b1_v[sl]
            a2_v[sl] = a2_v[sl] + b2_v[sl]
            return carry

        lax.fori_loop(0, _EB // 16, body, 0)
        pltpu.sync_copy(a1_v, s1_h.at[pl.ds(base, _EB)])
        pltpu.sync_copy(a2_v, s2_h.at[pl.ds(base, _EB)])

    return ek(u1, v1, u2, v2, epi, epj)


_NCH = 40
_CB = 128
_EP2 = 32 * _NCH * _CB
_RPT = 624


def _spmm_sc(x, src3, dst3, w3, k):
    mesh = plsc.VectorSubcoreMesh(core_axis_name="c", subcore_axis_name="s")

    @functools.partial(
        pl.kernel, mesh=mesh,
        out_type=jax.ShapeDtypeStruct((2, _N, k), jnp.float32),
        scratch_types=[
            pltpu.VMEM((_NCH, _CB), jnp.int32),
            pltpu.VMEM((_NCH, _CB), jnp.int32),
            pltpu.VMEM((_NCH, _CB), jnp.float32),
            pltpu.VMEM((2, _CB, k), jnp.float32),
            pltpu.VMEM_SHARED((_N, k), jnp.float32),
            pltpu.SemaphoreType.DMA,
            pltpu.SemaphoreType.DMA,
            pltpu.SemaphoreType.DMA,
            pltpu.SemaphoreType.DMA,
        ],
    )
    def sk(x_h, src_h, dst_h, w_h, z_h, out_h,
           src_v, dst_v, w_v, rows2_v, acc_sh,
           g0, g1, s0, s1):
        cid = lax.axis_index("c")
        sid = lax.axis_index("s")
        wid = cid * 16 + sid
        rbase = sid * _RPT
        pltpu.sync_copy(z_h.at[pl.ds(rbase, _RPT)],
                        acc_sh.at[pl.ds(rbase, _RPT)])

        @pl.when(sid == 15)
        def _zero_tail():
            pltpu.sync_copy(z_h.at[pl.ds(16 * _RPT, _N - 16 * _RPT)],
                            acc_sh.at[pl.ds(16 * _RPT, _N - 16 * _RPT)])
        pltpu.sync_copy(src_h.at[wid], src_v)
        pltpu.sync_copy(dst_h.at[wid], dst_v)
        pltpu.sync_copy(w_h.at[wid], w_v)
        plsc.subcore_barrier()

        gsems = (g0, g1)
        ssems = (s0, s1)

        def scale(rv, ci):
            def grp(g, c2):
                wv = w_v[ci, pl.ds(g * 16, 16)]
                for l in range(16):
                    wb = jnp.take(wv, jnp.full((16,), l, jnp.int32))
                    e = g * 16 + l
                    for kk in range(k // 16):
                        sl = pl.ds(kk * 16, 16)
                        rv[e, sl] = rv[e, sl] * wb
                return c2

            lax.fori_loop(0, _CB // 16, grp, 0)

        pltpu.async_copy(x_h.at[src_v.at[0]], rows2_v.at[0], gsems[0])

        def pipe(g, carry):
            for b in range(2):
                ci = 2 * g + b
                bn = 1 - b
                if b == 0:
                    @pl.when(g > 0)
                    def _dr0():
                        pltpu.make_async_copy(z_h.at[pl.ds(0, _CB)],
                                              rows2_v.at[bn], ssems[bn]).wait()

                    pltpu.async_copy(x_h.at[src_v.at[ci + 1]],
                                     rows2_v.at[bn], gsems[bn])
                else:
                    pltpu.make_async_copy(z_h.at[pl.ds(0, _CB)],
                                          rows2_v.at[bn], ssems[bn]).wait()

                    @pl.when(g < (_NCH // 2) - 1)
                    def _ng():
                        pltpu.async_copy(x_h.at[src_v.at[ci + 1]],
                                         rows2_v.at[bn], gsems[bn])

                pltpu.make_async_copy(x_h.at[src_v.at[ci]],
                                      rows2_v.at[b], gsems[b]).wait()
                scale(rows2_v.at[b], ci)
                pltpu.async_copy(rows2_v.at[b], acc_sh.at[dst_v.at[ci]],
                                 ssems[b], add=True)
            return carry

        lax.fori_loop(0, _NCH // 2, pipe, 0)
        pltpu.make_async_copy(z_h.at[pl.ds(0, _CB)],
                              rows2_v.at[(_NCH - 1) % 2],
                              ssems[(_NCH - 1) % 2]).wait()
        plsc.subcore_barrier()
        pltpu.sync_copy(acc_sh.at[pl.ds(rbase, _RPT)],
                        out_h.at[cid, pl.ds(rbase, _RPT)])

        @pl.when(sid == 15)
        def _out_tail():
            pltpu.sync_copy(acc_sh.at[pl.ds(16 * _RPT, _N - 16 * _RPT)],
                            out_h.at[cid, pl.ds(16 * _RPT, _N - 16 * _RPT)])

    parts = sk(x, src3, dst3, w3, jnp.zeros((_N, k), jnp.float32))
    return parts[0] + parts[1]


def kernel(features, edge_src, edge_dst, edge_w, labels, mask,
           edge_pos_i, edge_pos_j, train_idx, mat01_intra, mat01_inter,
           W0, b0, W1, b1, Wh0, bh0, Wh1, bh1, Wc, bc):
    epad = _EP2 - _E
    pad_idx = (jnp.arange(epad, dtype=jnp.int32) * 97) % _N
    src3 = jnp.concatenate([edge_src.astype(jnp.int32),
                            pad_idx]).reshape(32, _NCH, _CB)
    dst3 = jnp.concatenate([edge_dst.astype(jnp.int32),
                            pad_idx]).reshape(32, _NCH, _CB)
    w3 = jnp.pad(edge_w, (0, epad)).reshape(32, _NCH, _CB)
    sfeat = _spmm_sc(features, src3, dst3, w3, _D)
    h0cat = jax.nn.relu(sfeat @ jnp.concatenate([W0, Wh0], axis=1)
                        + jnp.concatenate([b0, bh0]))
    s1cat = _spmm_sc(h0cat, src3, dst3, w3, _D)
    cv_gcn = _l2n(s1cat[:, :_H] @ W1 + b1)
    cv_hg = _l2n(s1cat[:, _H:] @ Wh1 + bh1)
    outputs = _l2n(0.6 * cv_gcn + 0.4 * cv_hg)

    m = mask / jnp.mean(mask)
    logp = jax.nn.log_softmax(outputs, axis=1)
    loss_q = jnp.mean(-(labels * logp).sum(axis=1) * m)

    a = Wc[:_C, 0]
    b = Wc[_C:, 0]
    u1 = jnp.pad(cv_gcn @ a + bc[0], (0, _NP - _N))
    v1 = jnp.pad(cv_hg @ b, (0, _NP - _N))
    u2 = jnp.pad(cv_hg @ a + bc[0], (0, _NP - _N))
    v2 = jnp.pad(cv_gcn @ b, (0, _NP - _N))
    epi = jnp.pad(edge_pos_i.astype(jnp.int32), (0, _EP - _E))
    epj = jnp.pad(edge_pos_j.astype(jnp.int32), (0, _EP - _E))
    s1e, s2e = _edge_logits_sc(u1, v1, u2, v2, epi, epj)
    p1 = -jnp.mean(jnp.log(jax.nn.sigmoid(s1e[:_E])))
    p2 = -jnp.mean(jnp.log(jax.nn.sigmoid(s2e[:_E])))
    p_e_xy = p1 + p2

    pad = _NP - _N
    g2p = jnp.pad(2.0 * cv_gcn, ((0, pad), (0, 0))).astype(jnp.bfloat16)
    htp = jnp.pad(cv_hg.T, ((0, 0), (0, pad))).astype(jnp.bfloat16)
    rowsum, colsum = _nxn_reductions(g2p, htp)
    d = jnp.exp(2.0 * jnp.sum(cv_gcn * cv_hg, axis=1))
    pn1 = d / (rowsum / _N)
    pn2 = d / (colsum / _N)
    closs = -0.9 * jnp.mean(jnp.log(jnp.concatenate([pn1, pn2], axis=0)))

    h1s = cv_gcn[train_idx]
    h2s = cv_hg[train_idx]
    padt = _TP - _T
    h1p2 = jnp.pad(2.0 * h1s, ((0, padt), (0, 0)))
    h2tp = jnp.pad(h2s.T, ((0, 0), (0, padt)))
    intrap = jnp.pad(mat01_intra, ((0, padt), (0, padt)))
    intratp = jnp.pad(mat01_intra.T, ((0, padt), (0, padt)))
    sprow, rowtot, spcol, coltot = _sup_reductions(h1p2, h2tp, intrap, intratp)
    rowsum01 = jnp.sum(mat01_intra, axis=1)
    sup1 = (sprow / rowsum01) / (rowtot / (_T - 1))
    sup2 = (spcol / rowsum01) / (coltot / (_T - 1))
    closs = closs - 0.9 * jnp.mean(jnp.log(jnp.concatenate([sup1, sup2], axis=0)))

    total = loss_q + 0.4 * p_e_xy + closs
    for v in (W0, b0, W1, b1, Wc, bc):
        total = total + _WD * 0.5 * jnp.sum(v ** 2)

    acc = jnp.mean((jnp.argmax(outputs, axis=1) ==
                    jnp.argmax(labels, axis=1)).astype(jnp.float32) * m)
    return (outputs, total, acc)


# speedup vs baseline: 2.4409x; 1.0213x over previous
import functools

import jax
import jax.numpy as jnp
from jax import lax
from jax.experimental import pallas as pl
from jax.experimental.pallas import tpu as pltpu
from jax.experimental.pallas import tpu_sc as plsc

_N = 10000
_E = 160000
_D = 128
_H = 64
_C = 16
_T = 1000
_WD = 5e-4

_TBR = 2048
_TBC = 2048
_NP = 10240
_GI = _NP // _TBR
_GJ = _NP // _TBC

_TP = 1024
_RT = 128


def _l2n(x):
    n = jnp.sqrt(jnp.sum(x * x, axis=1, keepdims=True))
    return x / jnp.maximum(n, 1e-12)


def _nxn_body(g_ref, ht_ref, rowp_ref, colp_ref):
    s = jax.lax.dot_general(g_ref[...], ht_ref[...], (((1,), (0,)), ((), ())),
                            preferred_element_type=jnp.float32)
    p = jnp.exp(s)
    rowp_ref[...] = jnp.sum(p, axis=1, keepdims=True).reshape(1, _TBR, 1)
    colp_ref[...] = jnp.sum(p, axis=0, keepdims=True).reshape(1, 1, _TBC)


def _nxn_reductions(g2p, htp):
    rowp, colp = pl.pallas_call(
        _nxn_body,
        grid=(_GI, _GJ),
        in_specs=[
            pl.BlockSpec((_TBR, _C), lambda i, j: (i, 0)),
            pl.BlockSpec((_C, _TBC), lambda i, j: (0, j)),
        ],
        out_specs=[
            pl.BlockSpec((1, _TBR, 1), lambda i, j: (j, i, 0)),
            pl.BlockSpec((1, 1, _TBC), lambda i, j: (i, 0, j)),
        ],
        out_shape=[
            jax.ShapeDtypeStruct((_GJ, _NP, 1), jnp.float32),
            jax.ShapeDtypeStruct((_GI, 1, _NP), jnp.float32),
        ],
    )(g2p, htp)
    pad = _NP - _N
    rowsum = jnp.sum(rowp, axis=0)[:_N, 0] - pad
    colsum = jnp.sum(colp, axis=(0, 1))[:_N] - pad
    return rowsum, colsum


def _sup_body(h1_ref, h2t_ref, intra_ref, intrat_ref,
              sprow_ref, rowtot_ref, spcol_ref, coltot_ref):
    i = pl.program_id(0)
    s = jax.lax.dot_general(h1_ref[...], h2t_ref[...], (((1,), (0,)), ((), ())),
                            preferred_element_type=jnp.float32)
    p = jnp.exp(s)
    sprow_ref[...] = jnp.sum(p * intra_ref[...], axis=1, keepdims=True)
    rowtot_ref[...] = jnp.sum(p, axis=1, keepdims=True)

    @pl.when(i == 0)
    def _init():
        spcol_ref[...] = jnp.zeros_like(spcol_ref)
        coltot_ref[...] = jnp.zeros_like(coltot_ref)

    spcol_ref[...] += jnp.sum(p * intrat_ref[...], axis=0, keepdims=True)
    coltot_ref[...] += jnp.sum(p, axis=0, keepdims=True)


def _sup_reductions(h1p2, h2tp, intrap, intratp):
    grid = (_TP // _RT,)
    sprow, rowtot, spcol, coltot = pl.pallas_call(
        _sup_body,
        grid=grid,
        in_specs=[
            pl.BlockSpec((_RT, _C), lambda i: (i, 0)),
            pl.BlockSpec((_C, _TP), lambda i: (0, 0)),
            pl.BlockSpec((_RT, _TP), lambda i: (i, 0)),
            pl.BlockSpec((_RT, _TP), lambda i: (i, 0)),
        ],
        out_specs=[
            pl.BlockSpec((_RT, 1), lambda i: (i, 0)),
            pl.BlockSpec((_RT, 1), lambda i: (i, 0)),
            pl.BlockSpec((1, _TP), lambda i: (0, 0)),
            pl.BlockSpec((1, _TP), lambda i: (0, 0)),
        ],
        out_shape=[
            jax.ShapeDtypeStruct((_TP, 1), jnp.float32),
            jax.ShapeDtypeStruct((_TP, 1), jnp.float32),
            jax.ShapeDtypeStruct((1, _TP), jnp.float32),
            jax.ShapeDtypeStruct((1, _TP), jnp.float32),
        ],
    )(h1p2, h2tp, intrap, intratp)
    padt = _TP - _T
    return (sprow[:_T, 0], rowtot[:_T, 0] - padt,
            spcol[0, :_T], coltot[0, :_T] - padt)


_NW = 32
_EP = 160256
_EB = _EP // _NW


def _edge_logits_sc(u1, v1, u2, v2, epi, epj):
    mesh = plsc.VectorSubcoreMesh(core_axis_name="c", subcore_axis_name="s")

    @functools.partial(
        pl.kernel, mesh=mesh,
        out_type=[jax.ShapeDtypeStruct((_EP,), jnp.float32),
                  jax.ShapeDtypeStruct((_EP,), jnp.float32)],
        scratch_types=[pltpu.VMEM((_EB,), jnp.int32)] * 2
        + [pltpu.VMEM((_EB,), jnp.float32)] * 4
        + [pltpu.SemaphoreType.DMA],
    )
    def ek(u1_h, v1_h, u2_h, v2_h, epi_h, epj_h, s1_h, s2_h,
           ei_v, ej_v, a1_v, b1_v, a2_v, b2_v, sem):
        wid = lax.axis_index("s") * 2 + lax.axis_index("c")
        base = wid * _EB
        pltpu.sync_copy(epi_h.at[pl.ds(base, _EB)], ei_v)
        pltpu.sync_copy(epj_h.at[pl.ds(base, _EB)], ej_v)
        pltpu.async_copy(u1_h.at[ei_v], a1_v, sem)
        pltpu.async_copy(v1_h.at[ej_v], b1_v, sem)
        pltpu.async_copy(u2_h.at[ei_v], a2_v, sem)
        last = pltpu.async_copy(v2_h.at[ej_v], b2_v, sem)
        last.wait()
        last.wait()
        last.wait()
        last.wait()

        def body(k, carry):
            sl = pl.ds(k * 16, 16)
            a1_v[sl] = a1_v[sl] + b1_v[sl]
            a2_v[sl] = a2_v[sl] + b2_v[sl]
            return carry

        lax.fori_loop(0, _EB // 16, body, 0)
        pltpu.sync_copy(a1_v, s1_h.at[pl.ds(base, _EB)])
        pltpu.sync_copy(a2_v, s2_h.at[pl.ds(base, _EB)])

    return ek(u1, v1, u2, v2, epi, epj)


_NCH = 40
_CB = 128
_EP2 = 32 * _NCH * _CB
_RPT = 624


def _spmm_sc(x, src3, dst3, w3, k):
    mesh = plsc.VectorSubcoreMesh(core_axis_name="c", subcore_axis_name="s")

    @functools.partial(
        pl.kernel, mesh=mesh,
        out_type=jax.ShapeDtypeStruct((2, _N, k), jnp.float32),
        scratch_types=[
            pltpu.VMEM((_NCH, _CB), jnp.int32),
            pltpu.VMEM((_NCH, _CB), jnp.int32),
            pltpu.VMEM((_NCH, _CB), jnp.float32),
            pltpu.VMEM((2, _CB, k), jnp.float32),
            pltpu.VMEM_SHARED((_N, k), jnp.float32),
            pltpu.SemaphoreType.DMA,
            pltpu.SemaphoreType.DMA,
            pltpu.SemaphoreType.DMA,
            pltpu.SemaphoreType.DMA,
        ],
    )
    def sk(x_h, src_h, dst_h, w_h, z_h, out_h,
           src_v, dst_v, w_v, rows2_v, acc_sh,
           g0, g1, s0, s1):
        cid = lax.axis_index("c")
        sid = lax.axis_index("s")
        wid = cid * 16 + sid
        rbase = sid * _RPT
        pltpu.sync_copy(z_h.at[pl.ds(rbase, _RPT)],
                        acc_sh.at[pl.ds(rbase, _RPT)])

        @pl.when(sid == 15)
        def _zero_tail():
            pltpu.sync_copy(z_h.at[pl.ds(16 * _RPT, _N - 16 * _RPT)],
                            acc_sh.at[pl.ds(16 * _RPT, _N - 16 * _RPT)])
        pltpu.sync_copy(src_h.at[wid], src_v)
        pltpu.sync_copy(dst_h.at[wid], dst_v)
        pltpu.sync_copy(w_h.at[wid], w_v)
        plsc.subcore_barrier()

        gsems = (g0, g1)
        ssems = (s0, s1)

        def scale(rv, ci):
            def grp(g, c2):
                wv = w_v[ci, pl.ds(g * 16, 16)]
                for l in range(16):
                    wb = jnp.take(wv, jnp.full((16,), l, jnp.int32))
                    e = g * 16 + l
                    for kk in range(k // 16):
                        sl = pl.ds(kk * 16, 16)
                        rv[e, sl] = rv[e, sl] * wb
                return c2

            lax.fori_loop(0, _CB // 16, grp, 0)

        pltpu.async_copy(x_h.at[src_v.at[0]], rows2_v.at[0], gsems[0])

        def pipe(g, carry):
            for b in range(2):
                ci = 2 * g + b
                bn = 1 - b
                if b == 0:
                    @pl.when(g > 0)
                    def _dr0():
                        pltpu.make_async_copy(z_h.at[pl.ds(0, _CB)],
                                              rows2_v.at[bn], ssems[bn]).wait()

                    pltpu.async_copy(x_h.at[src_v.at[ci + 1]],
                                     rows2_v.at[bn], gsems[bn])
                else:
                    pltpu.make_async_copy(z_h.at[pl.ds(0, _CB)],
                                          rows2_v.at[bn], ssems[bn]).wait()

                    @pl.when(g < (_NCH // 2) - 1)
                    def _ng():
                        pltpu.async_copy(x_h.at[src_v.at[ci + 1]],
                                         rows2_v.at[bn], gsems[bn])

                pltpu.make_async_copy(x_h.at[src_v.at[ci]],
                                      rows2_v.at[b], gsems[b]).wait()
                scale(rows2_v.at[b], ci)
                pltpu.async_copy(rows2_v.at[b], acc_sh.at[dst_v.at[ci]],
                                 ssems[b], add=True)
            return carry

        lax.fori_loop(0, _NCH // 2, pipe, 0)
        pltpu.make_async_copy(z_h.at[pl.ds(0, _CB)],
                              rows2_v.at[(_NCH - 1) % 2],
                              ssems[(_NCH - 1) % 2]).wait()
        plsc.subcore_barrier()
        pltpu.sync_copy(acc_sh.at[pl.ds(rbase, _RPT)],
                        out_h.at[cid, pl.ds(rbase, _RPT)])

        @pl.when(sid == 15)
        def _out_tail():
            pltpu.sync_copy(acc_sh.at[pl.ds(16 * _RPT, _N - 16 * _RPT)],
                            out_h.at[cid, pl.ds(16 * _RPT, _N - 16 * _RPT)])

    parts = sk(x, src3, dst3, w3, jnp.zeros((_N, k), jnp.float32))
    return parts[0] + parts[1]


def kernel(features, edge_src, edge_dst, edge_w, labels, mask,
           edge_pos_i, edge_pos_j, train_idx, mat01_intra, mat01_inter,
           W0, b0, W1, b1, Wh0, bh0, Wh1, bh1, Wc, bc):
    epad = _EP2 - _E
    pad_idx = (jnp.arange(epad, dtype=jnp.int32) * 97) % _N
    src3 = jnp.concatenate([edge_src.astype(jnp.int32),
                            pad_idx]).reshape(32, _NCH, _CB)
    dst3 = jnp.concatenate([edge_dst.astype(jnp.int32),
                            pad_idx]).reshape(32, _NCH, _CB)
    w3 = jnp.pad(edge_w, (0, epad)).reshape(32, _NCH, _CB)
    sfeat = _spmm_sc(features, src3, dst3, w3, _D)
    h0cat = jax.nn.relu(sfeat @ jnp.concatenate([W0, Wh0], axis=1)
                        + jnp.concatenate([b0, bh0]))
    s1cat = _spmm_sc(h0cat, src3, dst3, w3, _D)
    cv_gcn = _l2n(s1cat[:, :_H] @ W1 + b1)
    cv_hg = _l2n(s1cat[:, _H:] @ Wh1 + bh1)
    outputs = _l2n(0.6 * cv_gcn + 0.4 * cv_hg)

    m = mask / jnp.mean(mask)
    logp = jax.nn.log_softmax(outputs, axis=1)
    loss_q = jnp.mean(-(labels * logp).sum(axis=1) * m)

    a = Wc[:_C, 0]
    b = Wc[_C:, 0]
    u1 = jnp.pad(cv_gcn @ a + bc[0], (0, _NP - _N))
    v1 = jnp.pad(cv_hg @ b, (0, _NP - _N))
    u2 = jnp.pad(cv_hg @ a + bc[0], (0, _NP - _N))
    v2 = jnp.pad(cv_gcn @ b, (0, _NP - _N))
    epi = jnp.pad(edge_pos_i.astype(jnp.int32), (0, _EP - _E))
    epj = jnp.pad(edge_pos_j.astype(jnp.int32), (0, _EP - _E))
    s1e, s2e = _edge_logits_sc(u1, v1, u2, v2, epi, epj)
    p1 = -jnp.mean(jnp.log(jax.nn.sigmoid(s1e[:_E])))
    p2 = -jnp.mean(jnp.log(jax.nn.sigmoid(s2e[:_E])))
    p_e_xy = p1 + p2

    pad = _NP - _N
    g2p = jnp.pad(2.0 * cv_gcn, ((0, pad), (0, 0))).astype(jnp.bfloat16)
    htp = jnp.pad(cv_hg.T, ((0, 0), (0, pad))).astype(jnp.bfloat16)
    rowsum, colsum = _nxn_reductions(g2p, htp)
    d = jnp.exp(2.0 * jnp.sum(cv_gcn * cv_hg, axis=1))
    pn1 = d / (rowsum / _N)
    pn2 = d / (colsum / _N)
    closs = -0.9 * jnp.mean(jnp.log(jnp.concatenate([pn1, pn2], axis=0)))

    h1s = cv_gcn[train_idx]
    h2s = cv_hg[train_idx]
    padt = _TP - _T
    h1p2 = jnp.pad(2.0 * h1s, ((0, padt), (0, 0)))
    h2tp = jnp.pad(h2s.T, ((0, 0), (0, padt)))
    intrap = jnp.pad(mat01_intra, ((0, padt), (0, padt)))
    intratp = jnp.pad(mat01_intra.T, ((0, padt), (0, padt)))
    sprow, rowtot, spcol, coltot = _sup_reductions(h1p2, h2tp, intrap, intratp)
    rowsum01 = jnp.sum(mat01_intra, axis=1)
    sup1 = (sprow / rowsum01) / (rowtot / (_T - 1))
    sup2 = (spcol / rowsum01) / (coltot / (_T - 1))
    closs = closs - 0.9 * jnp.mean(jnp.log(jnp.concatenate([sup1, sup2], axis=0)))

    total = loss_q + 0.4 * p_e_xy + closs
    for v in (W0, b0, W1, b1, Wc, bc):
        total = total + _WD * 0.5 * jnp.sum(v ** 2)

    acc = jnp.mean((jnp.argmax(outputs, axis=1) ==
                    jnp.argmax(labels, axis=1)).astype(jnp.float32) * m)
    return (outputs, total, acc)


# speedup vs baseline: 2.5315x; 1.0371x over previous
import functools

import jax
import jax.numpy as jnp
from jax import lax
from jax.experimental import pallas as pl
from jax.experimental.pallas import tpu as pltpu
from jax.experimental.pallas import tpu_sc as plsc

_N = 10000
_E = 160000
_D = 128
_H = 64
_C = 16
_T = 1000
_WD = 5e-4

_TBR = 2048
_TBC = 2048
_NP = 10240
_GI = _NP // _TBR
_GJ = _NP // _TBC

_TP = 1024
_RT = 128


def _l2n(x):
    n = jnp.sqrt(jnp.sum(x * x, axis=1, keepdims=True))
    return x / jnp.maximum(n, 1e-12)


def _nxn_body(g_ref, ht_ref, rowp_ref, colp_ref):
    s = jax.lax.dot_general(g_ref[...], ht_ref[...], (((1,), (0,)), ((), ())),
                            preferred_element_type=jnp.float32)
    p = jnp.exp(s)
    rowp_ref[...] = jnp.sum(p, axis=1, keepdims=True).reshape(1, _TBR, 1)
    colp_ref[...] = jnp.sum(p, axis=0, keepdims=True).reshape(1, 1, _TBC)


def _nxn_reductions(g2p, htp):
    rowp, colp = pl.pallas_call(
        _nxn_body,
        grid=(_GI, _GJ),
        in_specs=[
            pl.BlockSpec((_TBR, _C), lambda i, j: (i, 0)),
            pl.BlockSpec((_C, _TBC), lambda i, j: (0, j)),
        ],
        out_specs=[
            pl.BlockSpec((1, _TBR, 1), lambda i, j: (j, i, 0)),
            pl.BlockSpec((1, 1, _TBC), lambda i, j: (i, 0, j)),
        ],
        out_shape=[
            jax.ShapeDtypeStruct((_GJ, _NP, 1), jnp.float32),
            jax.ShapeDtypeStruct((_GI, 1, _NP), jnp.float32),
        ],
    )(g2p, htp)
    pad = _NP - _N
    rowsum = jnp.sum(rowp, axis=0)[:_N, 0] - pad
    colsum = jnp.sum(colp, axis=(0, 1))[:_N] - pad
    return rowsum, colsum


def _sup_body(h1_ref, h2t_ref, h2_ref, h1t_ref, intra_ref,
              sprow_ref, rowtot_ref, spcol_ref, coltot_ref):
    p = jnp.exp(jax.lax.dot_general(
        h1_ref[...], h2t_ref[...], (((1,), (0,)), ((), ())),
        preferred_element_type=jnp.float32))
    q = jnp.exp(jax.lax.dot_general(
        h2_ref[...], h1t_ref[...], (((1,), (0,)), ((), ())),
        preferred_element_type=jnp.float32))
    intra = intra_ref[...]
    sprow_ref[...] = jnp.sum(p * intra, axis=1, keepdims=True)
    rowtot_ref[...] = jnp.sum(p, axis=1, keepdims=True)
    spcol_ref[...] = jnp.sum(q * intra, axis=1, keepdims=True)
    coltot_ref[...] = jnp.sum(q, axis=1, keepdims=True)


def _sup_reductions(h1p2, h2tp, h2p2, h1tp, intrap):
    grid = (_TP // _RT,)
    sprow, rowtot, spcol, coltot = pl.pallas_call(
        _sup_body,
        grid=grid,
        in_specs=[
            pl.BlockSpec((_RT, _C), lambda i: (i, 0)),
            pl.BlockSpec((_C, _TP), lambda i: (0, 0)),
            pl.BlockSpec((_RT, _C), lambda i: (i, 0)),
            pl.BlockSpec((_C, _TP), lambda i: (0, 0)),
            pl.BlockSpec((_RT, _TP), lambda i: (i, 0)),
        ],
        out_specs=[
            pl.BlockSpec((_RT, 1), lambda i: (i, 0)),
            pl.BlockSpec((_RT, 1), lambda i: (i, 0)),
            pl.BlockSpec((_RT, 1), lambda i: (i, 0)),
            pl.BlockSpec((_RT, 1), lambda i: (i, 0)),
        ],
        out_shape=[
            jax.ShapeDtypeStruct((_TP, 1), jnp.float32),
            jax.ShapeDtypeStruct((_TP, 1), jnp.float32),
            jax.ShapeDtypeStruct((_TP, 1), jnp.float32),
            jax.ShapeDtypeStruct((_TP, 1), jnp.float32),
        ],
    )(h1p2, h2tp, h2p2, h1tp, intrap)
    padt = _TP - _T
    return (sprow[:_T, 0], rowtot[:_T, 0] - padt,
            spcol[:_T, 0], coltot[:_T, 0] - padt)


_NW = 32
_EP = 160256
_EB = _EP // _NW


def _edge_logits_sc(u1, v1, u2, v2, epi, epj):
    mesh = plsc.VectorSubcoreMesh(core_axis_name="c", subcore_axis_name="s")

    @functools.partial(
        pl.kernel, mesh=mesh,
        out_type=[jax.ShapeDtypeStruct((_EP,), jnp.float32),
                  jax.ShapeDtypeStruct((_EP,), jnp.float32)],
        scratch_types=[pltpu.VMEM((_EB,), jnp.int32)] * 2
        + [pltpu.VMEM((_EB,), jnp.float32)] * 4
        + [pltpu.SemaphoreType.DMA],
    )
    def ek(u1_h, v1_h, u2_h, v2_h, epi_h, epj_h, s1_h, s2_h,
           ei_v, ej_v, a1_v, b1_v, a2_v, b2_v, sem):
        wid = lax.axis_index("s") * 2 + lax.axis_index("c")
        base = wid * _EB
        pltpu.sync_copy(epi_h.at[pl.ds(base, _EB)], ei_v)
        pltpu.sync_copy(epj_h.at[pl.ds(base, _EB)], ej_v)
        pltpu.async_copy(u1_h.at[ei_v], a1_v, sem)
        pltpu.async_copy(v1_h.at[ej_v], b1_v, sem)
        pltpu.async_copy(u2_h.at[ei_v], a2_v, sem)
        last = pltpu.async_copy(v2_h.at[ej_v], b2_v, sem)
        last.wait()
        last.wait()
        last.wait()
        last.wait()

        def body(k, carry):
            sl = pl.ds(k * 16, 16)
            a1_v[sl] = a1_v[sl] + b1_v[sl]
            a2_v[sl] = a2_v[sl] + b2_v[sl]
            return carry

        lax.fori_loop(0, _EB // 16, body, 0)
        pltpu.sync_copy(a1_v, s1_h.at[pl.ds(base, _EB)])
        pltpu.sync_copy(a2_v, s2_h.at[pl.ds(base, _EB)])

    return ek(u1, v1, u2, v2, epi, epj)


_NCH = 40
_CB = 128
_EP2 = 32 * _NCH * _CB
_RPT = 624


def _spmm_sc(x, src3, dst3, w3, k):
    mesh = plsc.VectorSubcoreMesh(core_axis_name="c", subcore_axis_name="s")

    @functools.partial(
        pl.kernel, mesh=mesh,
        out_type=jax.ShapeDtypeStruct((2, _N, k), jnp.float32),
        scratch_types=[
            pltpu.VMEM((_NCH, _CB), jnp.int32),
            pltpu.VMEM((_NCH, _CB), jnp.int32),
            pltpu.VMEM((_NCH, _CB), jnp.float32),
            pltpu.VMEM((2, _CB, k), jnp.float32),
            pltpu.VMEM_SHARED((_N, k), jnp.float32),
            pltpu.SemaphoreType.DMA,
            pltpu.SemaphoreType.DMA,
            pltpu.SemaphoreType.DMA,
            pltpu.SemaphoreType.DMA,
        ],
    )
    def sk(x_h, src_h, dst_h, w_h, z_h, out_h,
           src_v, dst_v, w_v, rows2_v, acc_sh,
           g0, g1, s0, s1):
        cid = lax.axis_index("c")
        sid = lax.axis_index("s")
        wid = cid * 16 + sid
        rbase = sid * _RPT
        pltpu.sync_copy(z_h.at[pl.ds(rbase, _RPT)],
                        acc_sh.at[pl.ds(rbase, _RPT)])

        @pl.when(sid == 15)
        def _zero_tail():
            pltpu.sync_copy(z_h.at[pl.ds(16 * _RPT, _N - 16 * _RPT)],
                            acc_sh.at[pl.ds(16 * _RPT, _N - 16 * _RPT)])
        pltpu.sync_copy(src_h.at[wid], src_v)
        pltpu.sync_copy(dst_h.at[wid], dst_v)
        pltpu.sync_copy(w_h.at[wid], w_v)
        plsc.subcore_barrier()

        gsems = (g0, g1)
        ssems = (s0, s1)

        def scale(rv, ci):
            def grp(g, c2):
                wv = w_v[ci, pl.ds(g * 16, 16)]
                for l in range(16):
                    wb = jnp.take(wv, jnp.full((16,), l, jnp.int32))
                    e = g * 16 + l
                    for kk in range(k // 16):
                        sl = pl.ds(kk * 16, 16)
                        rv[e, sl] = rv[e, sl] * wb
                return c2

            lax.fori_loop(0, _CB // 16, grp, 0)

        pltpu.async_copy(x_h.at[src_v.at[0]], rows2_v.at[0], gsems[0])

        def pipe(g, carry):
            for b in range(2):
                ci = 2 * g + b
                bn = 1 - b
                if b == 0:
                    @pl.when(g > 0)
                    def _dr0():
                        pltpu.make_async_copy(z_h.at[pl.ds(0, _CB)],
                                              rows2_v.at[bn], ssems[bn]).wait()

                    pltpu.async_copy(x_h.at[src_v.at[ci + 1]],
                                     rows2_v.at[bn], gsems[bn])
                else:
                    pltpu.make_async_copy(z_h.at[pl.ds(0, _CB)],
                                          rows2_v.at[bn], ssems[bn]).wait()

                    @pl.when(g < (_NCH // 2) - 1)
                    def _ng():
                        pltpu.async_copy(x_h.at[src_v.at[ci + 1]],
                                         rows2_v.at[bn], gsems[bn])

                pltpu.make_async_copy(x_h.at[src_v.at[ci]],
                                      rows2_v.at[b], gsems[b]).wait()
                scale(rows2_v.at[b], ci)
                pltpu.async_copy(rows2_v.at[b], acc_sh.at[dst_v.at[ci]],
                                 ssems[b], add=True)
            return carry

        lax.fori_loop(0, _NCH // 2, pipe, 0)
        pltpu.make_async_copy(z_h.at[pl.ds(0, _CB)],
                              rows2_v.at[(_NCH - 1) % 2],
                              ssems[(_NCH - 1) % 2]).wait()
        plsc.subcore_barrier()
        pltpu.sync_copy(acc_sh.at[pl.ds(rbase, _RPT)],
                        out_h.at[cid, pl.ds(rbase, _RPT)])

        @pl.when(sid == 15)
        def _out_tail():
            pltpu.sync_copy(acc_sh.at[pl.ds(16 * _RPT, _N - 16 * _RPT)],
                            out_h.at[cid, pl.ds(16 * _RPT, _N - 16 * _RPT)])

    parts = sk(x, src3, dst3, w3, jnp.zeros((_N, k), jnp.float32))
    return parts[0] + parts[1]


def kernel(features, edge_src, edge_dst, edge_w, labels, mask,
           edge_pos_i, edge_pos_j, train_idx, mat01_intra, mat01_inter,
           W0, b0, W1, b1, Wh0, bh0, Wh1, bh1, Wc, bc):
    epad = _EP2 - _E
    pad_idx = (jnp.arange(epad, dtype=jnp.int32) * 97) % _N
    src3 = jnp.concatenate([edge_src.astype(jnp.int32),
                            pad_idx]).reshape(32, _NCH, _CB)
    dst3 = jnp.concatenate([edge_dst.astype(jnp.int32),
                            pad_idx]).reshape(32, _NCH, _CB)
    w3 = jnp.pad(edge_w, (0, epad)).reshape(32, _NCH, _CB)
    sfeat = _spmm_sc(features, src3, dst3, w3, _D)
    h0cat = jax.nn.relu(sfeat @ jnp.concatenate([W0, Wh0], axis=1)
                        + jnp.concatenate([b0, bh0]))
    s1cat = _spmm_sc(h0cat, src3, dst3, w3, _D)
    cv_gcn = _l2n(s1cat[:, :_H] @ W1 + b1)
    cv_hg = _l2n(s1cat[:, _H:] @ Wh1 + bh1)
    outputs = _l2n(0.6 * cv_gcn + 0.4 * cv_hg)

    m = mask / jnp.mean(mask)
    logp = jax.nn.log_softmax(outputs, axis=1)
    loss_q = jnp.mean(-(labels * logp).sum(axis=1) * m)

    a = Wc[:_C, 0]
    b = Wc[_C:, 0]
    u1 = jnp.pad(cv_gcn @ a + bc[0], (0, _NP - _N))
    v1 = jnp.pad(cv_hg @ b, (0, _NP - _N))
    u2 = jnp.pad(cv_hg @ a + bc[0], (0, _NP - _N))
    v2 = jnp.pad(cv_gcn @ b, (0, _NP - _N))
    epi = jnp.pad(edge_pos_i.astype(jnp.int32), (0, _EP - _E))
    epj = jnp.pad(edge_pos_j.astype(jnp.int32), (0, _EP - _E))
    s1e, s2e = _edge_logits_sc(u1, v1, u2, v2, epi, epj)
    p1 = -jnp.mean(jnp.log(jax.nn.sigmoid(s1e[:_E])))
    p2 = -jnp.mean(jnp.log(jax.nn.sigmoid(s2e[:_E])))
    p_e_xy = p1 + p2

    pad = _NP - _N
    g2p = jnp.pad(2.0 * cv_gcn, ((0, pad), (0, 0))).astype(jnp.bfloat16)
    htp = jnp.pad(cv_hg.T, ((0, 0), (0, pad))).astype(jnp.bfloat16)
    rowsum, colsum = _nxn_reductions(g2p, htp)
    d = jnp.exp(2.0 * jnp.sum(cv_gcn * cv_hg, axis=1))
    pn1 = d / (rowsum / _N)
    pn2 = d / (colsum / _N)
    closs = -0.9 * jnp.mean(jnp.log(jnp.concatenate([pn1, pn2], axis=0)))

    h1s = cv_gcn[train_idx]
    h2s = cv_hg[train_idx]
    padt = _TP - _T
    h1p2 = jnp.pad(2.0 * h1s, ((0, padt), (0, 0)))
    h2tp = jnp.pad(h2s.T, ((0, 0), (0, padt)))
    h2p2 = jnp.pad(2.0 * h2s, ((0, padt), (0, 0)))
    h1tp = jnp.pad(h1s.T, ((0, 0), (0, padt)))
    intrap = jnp.pad(mat01_intra, ((0, padt), (0, padt)))
    sprow, rowtot, spcol, coltot = _sup_reductions(h1p2, h2tp, h2p2, h1tp,
                                                   intrap)
    rowsum01 = jnp.sum(mat01_intra, axis=1)
    sup1 = (sprow / rowsum01) / (rowtot / (_T - 1))
    sup2 = (spcol / rowsum01) / (coltot / (_T - 1))
    closs = closs - 0.9 * jnp.mean(jnp.log(jnp.concatenate([sup1, sup2], axis=0)))

    total = loss_q + 0.4 * p_e_xy + closs
    for v in (W0, b0, W1, b1, Wc, bc):
        total = total + _WD * 0.5 * jnp.sum(v ** 2)

    acc = jnp.mean((jnp.argmax(outputs, axis=1) ==
                    jnp.argmax(labels, axis=1)).astype(jnp.float32) * m)
    return (outputs, total, acc)


# speedup vs baseline: 2.5614x; 1.0118x over previous
import functools

import jax
import jax.numpy as jnp
from jax import lax
from jax.experimental import pallas as pl
from jax.experimental.pallas import tpu as pltpu
from jax.experimental.pallas import tpu_sc as plsc

_N = 10000
_E = 160000
_D = 128
_H = 64
_C = 16
_T = 1000
_WD = 5e-4

_TBR = 2048
_TBC = 2560
_NP = 10240
_GI = _NP // _TBR
_GJ = _NP // _TBC

_TP = 1024
_RT = 128


def _l2n(x):
    n = jnp.sqrt(jnp.sum(x * x, axis=1, keepdims=True))
    return x / jnp.maximum(n, 1e-12)


def _nxn_body(g_ref, ht_ref, rowp_ref, colp_ref):
    s = jax.lax.dot_general(g_ref[...], ht_ref[...], (((1,), (0,)), ((), ())),
                            preferred_element_type=jnp.float32)
    p = jnp.exp(s)
    rowp_ref[...] = jnp.sum(p, axis=1, keepdims=True).reshape(1, _TBR, 1)
    colp_ref[...] = jnp.sum(p, axis=0, keepdims=True).reshape(1, 1, _TBC)


def _nxn_reductions(g2p, htp):
    rowp, colp = pl.pallas_call(
        _nxn_body,
        grid=(_GI, _GJ),
        in_specs=[
            pl.BlockSpec((_TBR, _C), lambda i, j: (i, 0)),
            pl.BlockSpec((_C, _TBC), lambda i, j: (0, j)),
        ],
        out_specs=[
            pl.BlockSpec((1, _TBR, 1), lambda i, j: (j, i, 0)),
            pl.BlockSpec((1, 1, _TBC), lambda i, j: (i, 0, j)),
        ],
        out_shape=[
            jax.ShapeDtypeStruct((_GJ, _NP, 1), jnp.float32),
            jax.ShapeDtypeStruct((_GI, 1, _NP), jnp.float32),
        ],
    )(g2p, htp)
    pad = _NP - _N
    rowsum = jnp.sum(rowp, axis=0)[:_N, 0] - pad
    colsum = jnp.sum(colp, axis=(0, 1))[:_N] - pad
    return rowsum, colsum


def _sup_body(h1_ref, h2t_ref, h2_ref, h1t_ref, intra_ref,
              sprow_ref, rowtot_ref, spcol_ref, coltot_ref):
    p = jnp.exp(jax.lax.dot_general(
        h1_ref[...], h2t_ref[...], (((1,), (0,)), ((), ())),
        preferred_element_type=jnp.float32))
    q = jnp.exp(jax.lax.dot_general(
        h2_ref[...], h1t_ref[...], (((1,), (0,)), ((), ())),
        preferred_element_type=jnp.float32))
    intra = intra_ref[...]
    sprow_ref[...] = jnp.sum(p * intra, axis=1, keepdims=True)
    rowtot_ref[...] = jnp.sum(p, axis=1, keepdims=True)
    spcol_ref[...] = jnp.sum(q * intra, axis=1, keepdims=True)
    coltot_ref[...] = jnp.sum(q, axis=1, keepdims=True)


def _sup_reductions(h1p2, h2tp, h2p2, h1tp, intrap):
    grid = (_TP // _RT,)
    sprow, rowtot, spcol, coltot = pl.pallas_call(
        _sup_body,
        grid=grid,
        in_specs=[
            pl.BlockSpec((_RT, _C), lambda i: (i, 0)),
            pl.BlockSpec((_C, _TP), lambda i: (0, 0)),
            pl.BlockSpec((_RT, _C), lambda i: (i, 0)),
            pl.BlockSpec((_C, _TP), lambda i: (0, 0)),
            pl.BlockSpec((_RT, _TP), lambda i: (i, 0)),
        ],
        out_specs=[
            pl.BlockSpec((_RT, 1), lambda i: (i, 0)),
            pl.BlockSpec((_RT, 1), lambda i: (i, 0)),
            pl.BlockSpec((_RT, 1), lambda i: (i, 0)),
            pl.BlockSpec((_RT, 1), lambda i: (i, 0)),
        ],
        out_shape=[
            jax.ShapeDtypeStruct((_TP, 1), jnp.float32),
            jax.ShapeDtypeStruct((_TP, 1), jnp.float32),
            jax.ShapeDtypeStruct((_TP, 1), jnp.float32),
            jax.ShapeDtypeStruct((_TP, 1), jnp.float32),
        ],
    )(h1p2, h2tp, h2p2, h1tp, intrap)
    padt = _TP - _T
    return (sprow[:_T, 0], rowtot[:_T, 0] - padt,
            spcol[:_T, 0], coltot[:_T, 0] - padt)


_NW = 32
_EP = 160256
_EB = _EP // _NW


def _edge_logits_sc(u1, v1, u2, v2, epi, epj):
    mesh = plsc.VectorSubcoreMesh(core_axis_name="c", subcore_axis_name="s")

    @functools.partial(
        pl.kernel, mesh=mesh,
        out_type=[jax.ShapeDtypeStruct((_EP,), jnp.float32),
                  jax.ShapeDtypeStruct((_EP,), jnp.float32)],
        scratch_types=[pltpu.VMEM((_EB,), jnp.int32)] * 2
        + [pltpu.VMEM((_EB,), jnp.float32)] * 4
        + [pltpu.SemaphoreType.DMA],
    )
    def ek(u1_h, v1_h, u2_h, v2_h, epi_h, epj_h, s1_h, s2_h,
           ei_v, ej_v, a1_v, b1_v, a2_v, b2_v, sem):
        wid = lax.axis_index("s") * 2 + lax.axis_index("c")
        base = wid * _EB
        pltpu.sync_copy(epi_h.at[pl.ds(base, _EB)], ei_v)
        pltpu.sync_copy(epj_h.at[pl.ds(base, _EB)], ej_v)
        pltpu.async_copy(u1_h.at[ei_v], a1_v, sem)
        pltpu.async_copy(v1_h.at[ej_v], b1_v, sem)
        pltpu.async_copy(u2_h.at[ei_v], a2_v, sem)
        last = pltpu.async_copy(v2_h.at[ej_v], b2_v, sem)
        last.wait()
        last.wait()
        last.wait()
        last.wait()

        def body(k, carry):
            sl = pl.ds(k * 16, 16)
            a1_v[sl] = a1_v[sl] + b1_v[sl]
            a2_v[sl] = a2_v[sl] + b2_v[sl]
            return carry

        lax.fori_loop(0, _EB // 16, body, 0)
        pltpu.sync_copy(a1_v, s1_h.at[pl.ds(base, _EB)])
        pltpu.sync_copy(a2_v, s2_h.at[pl.ds(base, _EB)])

    return ek(u1, v1, u2, v2, epi, epj)


_NCH = 40
_CB = 128
_EP2 = 32 * _NCH * _CB
_RPT = 624


def _spmm_sc(x, src3, dst3, w3, k):
    mesh = plsc.VectorSubcoreMesh(core_axis_name="c", subcore_axis_name="s")

    @functools.partial(
        pl.kernel, mesh=mesh,
        out_type=jax.ShapeDtypeStruct((2, _N, k), jnp.float32),
        scratch_types=[
            pltpu.VMEM((_NCH, _CB), jnp.int32),
            pltpu.VMEM((_NCH, _CB), jnp.int32),
            pltpu.VMEM((_NCH, _CB), jnp.float32),
            pltpu.VMEM((2, _CB, k), jnp.float32),
            pltpu.VMEM_SHARED((_N, k), jnp.float32),
            pltpu.SemaphoreType.DMA,
            pltpu.SemaphoreType.DMA,
            pltpu.SemaphoreType.DMA,
            pltpu.SemaphoreType.DMA,
        ],
    )
    def sk(x_h, src_h, dst_h, w_h, z_h, out_h,
           src_v, dst_v, w_v, rows2_v, acc_sh,
           g0, g1, s0, s1):
        cid = lax.axis_index("c")
        sid = lax.axis_index("s")
        wid = cid * 16 + sid
        rbase = sid * _RPT
        pltpu.sync_copy(z_h.at[pl.ds(rbase, _RPT)],
                        acc_sh.at[pl.ds(rbase, _RPT)])

        @pl.when(sid == 15)
        def _zero_tail():
            pltpu.sync_copy(z_h.at[pl.ds(16 * _RPT, _N - 16 * _RPT)],
                            acc_sh.at[pl.ds(16 * _RPT, _N - 16 * _RPT)])
        pltpu.sync_copy(src_h.at[wid], src_v)
        pltpu.sync_copy(dst_h.at[wid], dst_v)
        pltpu.sync_copy(w_h.at[wid], w_v)
        plsc.subcore_barrier()

        gsems = (g0, g1)
        ssems = (s0, s1)

        def scale(rv, ci):
            def grp(g, c2):
                wv = w_v[ci, pl.ds(g * 16, 16)]
                for l in range(16):
                    wb = jnp.take(wv, jnp.full((16,), l, jnp.int32))
                    e = g * 16 + l
                    for kk in range(k // 16):
                        sl = pl.ds(kk * 16, 16)
                        rv[e, sl] = rv[e, sl] * wb
                return c2

            lax.fori_loop(0, _CB // 16, grp, 0)

        pltpu.async_copy(x_h.at[src_v.at[0]], rows2_v.at[0], gsems[0])

        def pipe(g, carry):
            for b in range(2):
                ci = 2 * g + b
                bn = 1 - b
                if b == 0:
                    @pl.when(g > 0)
                    def _dr0():
                        pltpu.make_async_copy(z_h.at[pl.ds(0, _CB)],
                                              rows2_v.at[bn], ssems[bn]).wait()

                    pltpu.async_copy(x_h.at[src_v.at[ci + 1]],
                                     rows2_v.at[bn], gsems[bn])
                else:
                    pltpu.make_async_copy(z_h.at[pl.ds(0, _CB)],
                                          rows2_v.at[bn], ssems[bn]).wait()

                    @pl.when(g < (_NCH // 2) - 1)
                    def _ng():
                        pltpu.async_copy(x_h.at[src_v.at[ci + 1]],
                                         rows2_v.at[bn], gsems[bn])

                pltpu.make_async_copy(x_h.at[src_v.at[ci]],
                                      rows2_v.at[b], gsems[b]).wait()
                scale(rows2_v.at[b], ci)
                pltpu.async_copy(rows2_v.at[b], acc_sh.at[dst_v.at[ci]],
                                 ssems[b], add=True)
            return carry

        lax.fori_loop(0, _NCH // 2, pipe, 0)
        pltpu.make_async_copy(z_h.at[pl.ds(0, _CB)],
                              rows2_v.at[(_NCH - 1) % 2],
                              ssems[(_NCH - 1) % 2]).wait()
        plsc.subcore_barrier()
        pltpu.sync_copy(acc_sh.at[pl.ds(rbase, _RPT)],
                        out_h.at[cid, pl.ds(rbase, _RPT)])

        @pl.when(sid == 15)
        def _out_tail():
            pltpu.sync_copy(acc_sh.at[pl.ds(16 * _RPT, _N - 16 * _RPT)],
                            out_h.at[cid, pl.ds(16 * _RPT, _N - 16 * _RPT)])

    parts = sk(x, src3, dst3, w3, jnp.zeros((_N, k), jnp.float32))
    return parts[0] + parts[1]


def kernel(features, edge_src, edge_dst, edge_w, labels, mask,
           edge_pos_i, edge_pos_j, train_idx, mat01_intra, mat01_inter,
           W0, b0, W1, b1, Wh0, bh0, Wh1, bh1, Wc, bc):
    epad = _EP2 - _E
    pad_idx = (jnp.arange(epad, dtype=jnp.int32) * 97) % _N
    src3 = jnp.concatenate([edge_src.astype(jnp.int32),
                            pad_idx]).reshape(32, _NCH, _CB)
    dst3 = jnp.concatenate([edge_dst.astype(jnp.int32),
                            pad_idx]).reshape(32, _NCH, _CB)
    w3 = jnp.pad(edge_w, (0, epad)).reshape(32, _NCH, _CB)
    sfeat = _spmm_sc(features, src3, dst3, w3, _D)
    h0cat = jax.nn.relu(sfeat @ jnp.concatenate([W0, Wh0], axis=1)
                        + jnp.concatenate([b0, bh0]))
    s1cat = _spmm_sc(h0cat, src3, dst3, w3, _D)
    cv_gcn = _l2n(s1cat[:, :_H] @ W1 + b1)
    cv_hg = _l2n(s1cat[:, _H:] @ Wh1 + bh1)
    outputs = _l2n(0.6 * cv_gcn + 0.4 * cv_hg)

    m = mask / jnp.mean(mask)
    logp = jax.nn.log_softmax(outputs, axis=1)
    loss_q = jnp.mean(-(labels * logp).sum(axis=1) * m)

    a = Wc[:_C, 0]
    b = Wc[_C:, 0]
    u1 = jnp.pad(cv_gcn @ a + bc[0], (0, _NP - _N))
    v1 = jnp.pad(cv_hg @ b, (0, _NP - _N))
    u2 = jnp.pad(cv_hg @ a + bc[0], (0, _NP - _N))
    v2 = jnp.pad(cv_gcn @ b, (0, _NP - _N))
    epi = jnp.pad(edge_pos_i.astype(jnp.int32), (0, _EP - _E))
    epj = jnp.pad(edge_pos_j.astype(jnp.int32), (0, _EP - _E))
    s1e, s2e = _edge_logits_sc(u1, v1, u2, v2, epi, epj)
    p1 = -jnp.mean(jnp.log(jax.nn.sigmoid(s1e[:_E])))
    p2 = -jnp.mean(jnp.log(jax.nn.sigmoid(s2e[:_E])))
    p_e_xy = p1 + p2

    pad = _NP - _N
    g2p = jnp.pad(2.0 * cv_gcn, ((0, pad), (0, 0))).astype(jnp.bfloat16)
    htp = jnp.pad(cv_hg.T, ((0, 0), (0, pad))).astype(jnp.bfloat16)
    rowsum, colsum = _nxn_reductions(g2p, htp)
    d = jnp.exp(2.0 * jnp.sum(cv_gcn * cv_hg, axis=1))
    pn1 = d / (rowsum / _N)
    pn2 = d / (colsum / _N)
    closs = -0.9 * jnp.mean(jnp.log(jnp.concatenate([pn1, pn2], axis=0)))

    h1s = cv_gcn[train_idx]
    h2s = cv_hg[train_idx]
    padt = _TP - _T
    h1p2 = jnp.pad(2.0 * h1s, ((0, padt), (0, 0)))
    h2tp = jnp.pad(h2s.T, ((0, 0), (0, padt)))
    h2p2 = jnp.pad(2.0 * h2s, ((0, padt), (0, 0)))
    h1tp = jnp.pad(h1s.T, ((0, 0), (0, padt)))
    intrap = jnp.pad(mat01_intra, ((0, padt), (0, padt)))
    sprow, rowtot, spcol, coltot = _sup_reductions(h1p2, h2tp, h2p2, h1tp,
                                                   intrap)
    rowsum01 = jnp.sum(mat01_intra, axis=1)
    sup1 = (sprow / rowsum01) / (rowtot / (_T - 1))
    sup2 = (spcol / rowsum01) / (coltot / (_T - 1))
    closs = closs - 0.9 * jnp.mean(jnp.log(jnp.concatenate([sup1, sup2], axis=0)))

    total = loss_q + 0.4 * p_e_xy + closs
    for v in (W0, b0, W1, b1, Wc, bc):
        total = total + _WD * 0.5 * jnp.sum(v ** 2)

    acc = jnp.mean((jnp.argmax(outputs, axis=1) ==
                    jnp.argmax(labels, axis=1)).astype(jnp.float32) * m)
    return (outputs, total, acc)


# speedup vs baseline: 2.5667x; 1.0021x over previous
import functools

import jax
import jax.numpy as jnp
from jax import lax
from jax.experimental import pallas as pl
from jax.experimental.pallas import tpu as pltpu
from jax.experimental.pallas import tpu_sc as plsc

_N = 10000
_E = 160000
_D = 128
_H = 64
_C = 16
_T = 1000
_WD = 5e-4

_TBR = 2560
_TBC = 2560
_NP = 10240
_GI = _NP // _TBR
_GJ = _NP // _TBC

_TP = 1024
_RT = 128


def _l2n(x):
    n = jnp.sqrt(jnp.sum(x * x, axis=1, keepdims=True))
    return x / jnp.maximum(n, 1e-12)


def _nxn_body(g_ref, ht_ref, rowp_ref, colp_ref):
    s = jax.lax.dot_general(g_ref[...], ht_ref[...], (((1,), (0,)), ((), ())),
                            preferred_element_type=jnp.float32)
    p = jnp.exp(s)
    rowp_ref[...] = jnp.sum(p, axis=1, keepdims=True).reshape(1, _TBR, 1)
    colp_ref[...] = jnp.sum(p, axis=0, keepdims=True).reshape(1, 1, _TBC)


def _nxn_reductions(g2p, htp):
    rowp, colp = pl.pallas_call(
        _nxn_body,
        grid=(_GI, _GJ),
        in_specs=[
            pl.BlockSpec((_TBR, _C), lambda i, j: (i, 0)),
            pl.BlockSpec((_C, _TBC), lambda i, j: (0, j)),
        ],
        out_specs=[
            pl.BlockSpec((1, _TBR, 1), lambda i, j: (j, i, 0)),
            pl.BlockSpec((1, 1, _TBC), lambda i, j: (i, 0, j)),
        ],
        out_shape=[
            jax.ShapeDtypeStruct((_GJ, _NP, 1), jnp.float32),
            jax.ShapeDtypeStruct((_GI, 1, _NP), jnp.float32),
        ],
    )(g2p, htp)
    pad = _NP - _N
    rowsum = jnp.sum(rowp, axis=0)[:_N, 0] - pad
    colsum = jnp.sum(colp, axis=(0, 1))[:_N] - pad
    return rowsum, colsum


def _sup_body(h1_ref, h2t_ref, h2_ref, h1t_ref, intra_ref,
              sprow_ref, rowtot_ref, spcol_ref, coltot_ref):
    p = jnp.exp(jax.lax.dot_general(
        h1_ref[...], h2t_ref[...], (((1,), (0,)), ((), ())),
        preferred_element_type=jnp.float32))
    q = jnp.exp(jax.lax.dot_general(
        h2_ref[...], h1t_ref[...], (((1,), (0,)), ((), ())),
        preferred_element_type=jnp.float32))
    intra = intra_ref[...]
    sprow_ref[...] = jnp.sum(p * intra, axis=1, keepdims=True)
    rowtot_ref[...] = jnp.sum(p, axis=1, keepdims=True)
    spcol_ref[...] = jnp.sum(q * intra, axis=1, keepdims=True)
    coltot_ref[...] = jnp.sum(q, axis=1, keepdims=True)


def _sup_reductions(h1p2, h2tp, h2p2, h1tp, intrap):
    grid = (_TP // _RT,)
    sprow, rowtot, spcol, coltot = pl.pallas_call(
        _sup_body,
        grid=grid,
        in_specs=[
            pl.BlockSpec((_RT, _C), lambda i: (i, 0)),
            pl.BlockSpec((_C, _TP), lambda i: (0, 0)),
            pl.BlockSpec((_RT, _C), lambda i: (i, 0)),
            pl.BlockSpec((_C, _TP), lambda i: (0, 0)),
            pl.BlockSpec((_RT, _TP), lambda i: (i, 0)),
        ],
        out_specs=[
            pl.BlockSpec((_RT, 1), lambda i: (i, 0)),
            pl.BlockSpec((_RT, 1), lambda i: (i, 0)),
            pl.BlockSpec((_RT, 1), lambda i: (i, 0)),
            pl.BlockSpec((_RT, 1), lambda i: (i, 0)),
        ],
        out_shape=[
            jax.ShapeDtypeStruct((_TP, 1), jnp.float32),
            jax.ShapeDtypeStruct((_TP, 1), jnp.float32),
            jax.ShapeDtypeStruct((_TP, 1), jnp.float32),
            jax.ShapeDtypeStruct((_TP, 1), jnp.float32),
        ],
    )(h1p2, h2tp, h2p2, h1tp, intrap)
    padt = _TP - _T
    return (sprow[:_T, 0], rowtot[:_T, 0] - padt,
            spcol[:_T, 0], coltot[:_T, 0] - padt)


_NW = 32
_EP = 160256
_EB = _EP // _NW


def _edge_logits_sc(u1, v1, u2, v2, epi, epj):
    mesh = plsc.VectorSubcoreMesh(core_axis_name="c", subcore_axis_name="s")

    @functools.partial(
        pl.kernel, mesh=mesh,
        out_type=[jax.ShapeDtypeStruct((_EP,), jnp.float32),
                  jax.ShapeDtypeStruct((_EP,), jnp.float32)],
        scratch_types=[pltpu.VMEM((_EB,), jnp.int32)] * 2
        + [pltpu.VMEM((_EB,), jnp.float32)] * 4
        + [pltpu.SemaphoreType.DMA],
    )
    def ek(u1_h, v1_h, u2_h, v2_h, epi_h, epj_h, s1_h, s2_h,
           ei_v, ej_v, a1_v, b1_v, a2_v, b2_v, sem):
        wid = lax.axis_index("s") * 2 + lax.axis_index("c")
        base = wid * _EB
        pltpu.sync_copy(epi_h.at[pl.ds(base, _EB)], ei_v)
        pltpu.sync_copy(epj_h.at[pl.ds(base, _EB)], ej_v)
        pltpu.async_copy(u1_h.at[ei_v], a1_v, sem)
        pltpu.async_copy(v1_h.at[ej_v], b1_v, sem)
        pltpu.async_copy(u2_h.at[ei_v], a2_v, sem)
        last = pltpu.async_copy(v2_h.at[ej_v], b2_v, sem)
        last.wait()
        last.wait()
        last.wait()
        last.wait()

        def body(k, carry):
            sl = pl.ds(k * 16, 16)
            a1_v[sl] = a1_v[sl] + b1_v[sl]
            a2_v[sl] = a2_v[sl] + b2_v[sl]
            return carry

        lax.fori_loop(0, _EB // 16, body, 0)
        pltpu.sync_copy(a1_v, s1_h.at[pl.ds(base, _EB)])
        pltpu.sync_copy(a2_v, s2_h.at[pl.ds(base, _EB)])

    return ek(u1, v1, u2, v2, epi, epj)


_NCH = 40
_CB = 128
_EP2 = 32 * _NCH * _CB
_RPT = 624


def _spmm_sc(x, src3, dst3, w3, k):
    mesh = plsc.VectorSubcoreMesh(core_axis_name="c", subcore_axis_name="s")

    @functools.partial(
        pl.kernel, mesh=mesh,
        out_type=jax.ShapeDtypeStruct((2, _N, k), jnp.float32),
        scratch_types=[
            pltpu.VMEM((_NCH, _CB), jnp.int32),
            pltpu.VMEM((_NCH, _CB), jnp.int32),
            pltpu.VMEM((_NCH, _CB), jnp.float32),
            pltpu.VMEM((2, _CB, k), jnp.float32),
            pltpu.VMEM_SHARED((_N, k), jnp.float32),
            pltpu.SemaphoreType.DMA,
            pltpu.SemaphoreType.DMA,
            pltpu.SemaphoreType.DMA,
            pltpu.SemaphoreType.DMA,
        ],
    )
    def sk(x_h, src_h, dst_h, w_h, z_h, out_h,
           src_v, dst_v, w_v, rows2_v, acc_sh,
           g0, g1, s0, s1):
        cid = lax.axis_index("c")
        sid = lax.axis_index("s")
        wid = cid * 16 + sid
        rbase = sid * _RPT
        pltpu.sync_copy(z_h.at[pl.ds(rbase, _RPT)],
                        acc_sh.at[pl.ds(rbase, _RPT)])

        @pl.when(sid == 15)
        def _zero_tail():
            pltpu.sync_copy(z_h.at[pl.ds(16 * _RPT, _N - 16 * _RPT)],
                            acc_sh.at[pl.ds(16 * _RPT, _N - 16 * _RPT)])
        pltpu.sync_copy(src_h.at[wid], src_v)
        pltpu.sync_copy(dst_h.at[wid], dst_v)
        pltpu.sync_copy(w_h.at[wid], w_v)
        plsc.subcore_barrier()

        gsems = (g0, g1)
        ssems = (s0, s1)

        def scale(rv, ci):
            def grp(g, c2):
                wv = w_v[ci, pl.ds(g * 16, 16)]
                for l in range(16):
                    wb = jnp.take(wv, jnp.full((16,), l, jnp.int32))
                    e = g * 16 + l
                    for kk in range(k // 16):
                        sl = pl.ds(kk * 16, 16)
                        rv[e, sl] = rv[e, sl] * wb
                return c2

            lax.fori_loop(0, _CB // 16, grp, 0)

        pltpu.async_copy(x_h.at[src_v.at[0]], rows2_v.at[0], gsems[0])

        def pipe(g, carry):
            for b in range(2):
                ci = 2 * g + b
                bn = 1 - b
                if b == 0:
                    @pl.when(g > 0)
                    def _dr0():
                        pltpu.make_async_copy(z_h.at[pl.ds(0, _CB)],
                                              rows2_v.at[bn], ssems[bn]).wait()

                    pltpu.async_copy(x_h.at[src_v.at[ci + 1]],
                                     rows2_v.at[bn], gsems[bn])
                else:
                    pltpu.make_async_copy(z_h.at[pl.ds(0, _CB)],
                                          rows2_v.at[bn], ssems[bn]).wait()

                    @pl.when(g < (_NCH // 2) - 1)
                    def _ng():
                        pltpu.async_copy(x_h.at[src_v.at[ci + 1]],
                                         rows2_v.at[bn], gsems[bn])

                pltpu.make_async_copy(x_h.at[src_v.at[ci]],
                                      rows2_v.at[b], gsems[b]).wait()
                scale(rows2_v.at[b], ci)
                pltpu.async_copy(rows2_v.at[b], acc_sh.at[dst_v.at[ci]],
                                 ssems[b], add=True)
            return carry

        lax.fori_loop(0, _NCH // 2, pipe, 0)
        pltpu.make_async_copy(z_h.at[pl.ds(0, _CB)],
                              rows2_v.at[(_NCH - 1) % 2],
                              ssems[(_NCH - 1) % 2]).wait()
        plsc.subcore_barrier()
        pltpu.sync_copy(acc_sh.at[pl.ds(rbase, _RPT)],
                        out_h.at[cid, pl.ds(rbase, _RPT)])

        @pl.when(sid == 15)
        def _out_tail():
            pltpu.sync_copy(acc_sh.at[pl.ds(16 * _RPT, _N - 16 * _RPT)],
                            out_h.at[cid, pl.ds(16 * _RPT, _N - 16 * _RPT)])

    parts = sk(x, src3, dst3, w3, jnp.zeros((_N, k), jnp.float32))
    return parts[0] + parts[1]


def kernel(features, edge_src, edge_dst, edge_w, labels, mask,
           edge_pos_i, edge_pos_j, train_idx, mat01_intra, mat01_inter,
           W0, b0, W1, b1, Wh0, bh0, Wh1, bh1, Wc, bc):
    epad = _EP2 - _E
    pad_idx = (jnp.arange(epad, dtype=jnp.int32) * 97) % _N
    src3 = jnp.concatenate([edge_src.astype(jnp.int32),
                            pad_idx]).reshape(32, _NCH, _CB)
    dst3 = jnp.concatenate([edge_dst.astype(jnp.int32),
                            pad_idx]).reshape(32, _NCH, _CB)
    w3 = jnp.pad(edge_w, (0, epad)).reshape(32, _NCH, _CB)
    sfeat = _spmm_sc(features, src3, dst3, w3, _D)
    h0cat = jax.nn.relu(sfeat @ jnp.concatenate([W0, Wh0], axis=1)
                        + jnp.concatenate([b0, bh0]))
    s1cat = _spmm_sc(h0cat, src3, dst3, w3, _D)
    cv_gcn = _l2n(s1cat[:, :_H] @ W1 + b1)
    cv_hg = _l2n(s1cat[:, _H:] @ Wh1 + bh1)
    outputs = _l2n(0.6 * cv_gcn + 0.4 * cv_hg)

    m = mask / jnp.mean(mask)
    logp = jax.nn.log_softmax(outputs, axis=1)
    loss_q = jnp.mean(-(labels * logp).sum(axis=1) * m)

    a = Wc[:_C, 0]
    b = Wc[_C:, 0]
    u1 = jnp.pad(cv_gcn @ a + bc[0], (0, _NP - _N))
    v1 = jnp.pad(cv_hg @ b, (0, _NP - _N))
    u2 = jnp.pad(cv_hg @ a + bc[0], (0, _NP - _N))
    v2 = jnp.pad(cv_gcn @ b, (0, _NP - _N))
    epi = jnp.pad(edge_pos_i.astype(jnp.int32), (0, _EP - _E))
    epj = jnp.pad(edge_pos_j.astype(jnp.int32), (0, _EP - _E))
    s1e, s2e = _edge_logits_sc(u1, v1, u2, v2, epi, epj)
    p1 = -jnp.mean(jnp.log(jax.nn.sigmoid(s1e[:_E])))
    p2 = -jnp.mean(jnp.log(jax.nn.sigmoid(s2e[:_E])))
    p_e_xy = p1 + p2

    pad = _NP - _N
    g2p = jnp.pad(2.0 * cv_gcn, ((0, pad), (0, 0))).astype(jnp.bfloat16)
    htp = jnp.pad(cv_hg.T, ((0, 0), (0, pad))).astype(jnp.bfloat16)
    rowsum, colsum = _nxn_reductions(g2p, htp)
    d = jnp.exp(2.0 * jnp.sum(cv_gcn * cv_hg, axis=1))
    pn1 = d / (rowsum / _N)
    pn2 = d / (colsum / _N)
    closs = -0.9 * jnp.mean(jnp.log(jnp.concatenate([pn1, pn2], axis=0)))

    h1s = cv_gcn[train_idx]
    h2s = cv_hg[train_idx]
    padt = _TP - _T
    h1p2 = jnp.pad(2.0 * h1s, ((0, padt), (0, 0)))
    h2tp = jnp.pad(h2s.T, ((0, 0), (0, padt)))
    h2p2 = jnp.pad(2.0 * h2s, ((0, padt), (0, 0)))
    h1tp = jnp.pad(h1s.T, ((0, 0), (0, padt)))
    intrap = jnp.pad(mat01_intra, ((0, padt), (0, padt)))
    sprow, rowtot, spcol, coltot = _sup_reductions(h1p2, h2tp, h2p2, h1tp,
                                                   intrap)
    rowsum01 = jnp.sum(mat01_intra, axis=1)
    sup1 = (sprow / rowsum01) / (rowtot / (_T - 1))
    sup2 = (spcol / rowsum01) / (coltot / (_T - 1))
    closs = closs - 0.9 * jnp.mean(jnp.log(jnp.concatenate([sup1, sup2], axis=0)))

    total = loss_q + 0.4 * p_e_xy + closs
    for v in (W0, b0, W1, b1, Wc, bc):
        total = total + _WD * 0.5 * jnp.sum(v ** 2)

    acc = jnp.mean((jnp.argmax(outputs, axis=1) ==
                    jnp.argmax(labels, axis=1)).astype(jnp.float32) * m)
    return (outputs, total, acc)
